# baseline (speedup 1.0000x reference)
import functools

import jax
import jax.numpy as jnp
from jax import lax
from jax.experimental import pallas as pl
from jax.experimental.pallas import tpu as pltpu
from jax.experimental.pallas import tpu_sc as plsc

T = 2048
HID = 2048
E = 8
DFF = 768
EPS = 1e-6

TBR = 128
TBM = 256
NSLOT = 2 * T
NT = NSLOT // TBM + E
NTP = NT * TBM
TBC = 256
HIDW = HID // 2

NC = 2
NS = 16
NW = NC * NS
TOK_PER_W = T // NW
ROW_PER_W = NSLOT // NW
CH = 32


def _rms_x2(x, ln2):
    var = jnp.mean(x * x, axis=-1, keepdims=True)
    return x * jax.lax.rsqrt(var + EPS) * ln2


def _count_body(hid_ref, ln2_ref, wg_ref, x2_ref, topi_ref, counts_ref):
    t = pl.program_id(0)
    x = hid_ref[...]
    x2 = _rms_x2(x, ln2_ref[...])
    x2_ref[...] = x2.astype(jnp.bfloat16)

    logits = jax.lax.dot_general(wg_ref[...], x2, (((1,), (1,)), ((), ())),
                                 preferred_element_type=jnp.float32)
    probs = jax.nn.softmax(logits, axis=0)
    iota_e = jax.lax.broadcasted_iota(jnp.int32, (E, TBR), 0)
    m1 = jnp.max(probs, axis=0, keepdims=True)
    i1 = jnp.min(jnp.where(probs == m1, iota_e, E), axis=0, keepdims=True)
    sel1 = (iota_e == i1).astype(jnp.float32)
    probs_m = jnp.where(iota_e == i1, -jnp.inf, probs)
    m2 = jnp.max(probs_m, axis=0, keepdims=True)
    i2 = jnp.min(jnp.where(probs_m == m2, iota_e, E), axis=0, keepdims=True)
    sel2 = (iota_e == i2).astype(jnp.float32)
    topi_ref[...] = jnp.concatenate([i1, i2], axis=0)

    @pl.when(t == 0)
    def _init_counts():
        counts_ref[...] = jnp.zeros_like(counts_ref)

    counts_ref[...] += (jnp.sum(sel1, axis=1, keepdims=True)
                        + jnp.sum(sel2, axis=1, keepdims=True))


def _pos_body(topi_ref, counts_ref, pos_ref, te_ref, tv_ref,
              run_ref, off_ref):
    t = pl.program_id(0)
    iota_e = jax.lax.broadcasted_iota(jnp.int32, (E, TBR), 0)
    ti = topi_ref[...]
    sel1 = (iota_e == ti[0:1, :]).astype(jnp.float32)
    sel2 = (iota_e == ti[1:2, :]).astype(jnp.float32)

    @pl.when(t == 0)
    def _offsets():
        c = counts_ref[...]
        pc = jnp.floor((c + (TBM - 1)) / TBM) * TBM
        a = jax.lax.broadcasted_iota(jnp.int32, (E, E), 0)
        b = jax.lax.broadcasted_iota(jnp.int32, (E, E), 1)
        lte = (b < a).astype(jnp.float32)
        off = jax.lax.dot_general(lte, pc, (((1,), (0,)), ((), ())),
                                  preferred_element_type=jnp.float32)
        off_ref[...] = off
        run_ref[...] = jnp.zeros_like(run_ref)
        s = (jax.lax.broadcasted_iota(jnp.int32, (E, NT), 1) * TBM
             ).astype(jnp.float32)
        offb = jnp.broadcast_to(off, (E, NT))
        pcb = jnp.broadcast_to(pc, (E, NT))
        mask = ((s >= offb) & (s < offb + pcb)).astype(jnp.int32)
        eio = jax.lax.broadcasted_iota(jnp.int32, (E, NT), 0)
        anym = jnp.max(mask, axis=0, keepdims=True)
        te = jnp.sum(mask * eio, axis=0, keepdims=True)
        te_ref[...] = jnp.where(anym == 1, te, E - 1)
        tv_ref[...] = anym

    run = run_ref[...]
    off = off_ref[...]
    r = jax.lax.broadcasted_iota(jnp.int32, (TBR, TBR), 0)
    c = jax.lax.broadcasted_iota(jnp.int32, (TBR, TBR), 1)
    lt = (r < c).astype(jnp.float32)
    lc1 = jax.lax.dot_general(sel1, lt, (((1,), (0,)), ((), ())),
                              preferred_element_type=jnp.float32)
    pos0 = jnp.sum(sel1 * (off + run + lc1), axis=0, keepdims=True)
    run1 = run + jnp.sum(sel1, axis=1, keepdims=True)
    lc2 = jax.lax.dot_general(sel2, lt, (((1,), (0,)), ((), ())),
                              preferred_element_type=jnp.float32)
    pos1 = jnp.sum(sel2 * (off + run1 + lc2), axis=0, keepdims=True)
    run_ref[...] = run1 + jnp.sum(sel2, axis=1, keepdims=True)
    pos_ref[...] = jnp.concatenate([pos0, pos1], axis=0).astype(jnp.int32)


def _sc_scatter_body(x2_hbm, pos_hbm, xs_hbm, rows_v, idx0_v, idx1_v):
    wid = lax.axis_index("s") * NC + lax.axis_index("c")
    base = wid * TOK_PER_W
    for cch in range(TOK_PER_W // CH):
        tok0 = base + cch * CH
        pltpu.sync_copy(x2_hbm.at[pl.ds(tok0, CH)], rows_v)
        pltpu.sync_copy(pos_hbm.at[0, pl.ds(tok0, CH)], idx0_v)
        pltpu.sync_copy(pos_hbm.at[1, pl.ds(tok0, CH)], idx1_v)
        pltpu.sync_copy(rows_v, xs_hbm.at[idx0_v])
        pltpu.sync_copy(rows_v, xs_hbm.at[idx1_v])


def _gmm_body(te_ref, tv_ref, xs_ref, w1_ref, w2_ref, y_ref,
              w1b_ref, w2b_ref):
    i = pl.program_id(0)
    new_exp = jnp.logical_or(i == 0,
                             te_ref[i] != te_ref[jnp.maximum(i - 1, 0)])

    @pl.when(new_exp)
    def _cache_cast():
        w1b_ref[...] = w1_ref[0].astype(jnp.bfloat16)
        w2b_ref[...] = w2_ref[0].astype(jnp.bfloat16)

    @pl.when(tv_ref[i] == 1)
    def _compute():
        xb = xs_ref[...]
        h = jax.lax.dot_general(xb, w1b_ref[...],
                                (((1,), (1,)), ((), ())),
                                preferred_element_type=jnp.float32)
        h = (h * jax.nn.sigmoid(h)).astype(jnp.bfloat16)
        y = jax.lax.dot_general(h, w2b_ref[...],
                                (((1,), (1,)), ((), ())),
                                preferred_element_type=jnp.float32)
        y_ref[...] = y.astype(jnp.bfloat16)


def _sc_gather_body(ys_hbm, posf_hbm, yg_hbm, rows_v, idx_v, sem):
    wid = lax.axis_index("s") * NC + lax.axis_index("c")
    base = wid * ROW_PER_W
    for cch in range(ROW_PER_W // CH):
        r0 = base + cch * CH
        pltpu.sync_copy(posf_hbm.at[pl.ds(r0, CH)], idx_v)
        pltpu.async_copy(ys_hbm.at[idx_v], rows_v, sem).wait()
        pltpu.sync_copy(rows_v, yg_hbm.at[pl.ds(r0, CH)])


def _combine_body(hid_ref, y0_ref, y1_ref, ln2_ref, wg_ref, wse_ref, wsd_ref,
                  out_ref):
    x = hid_ref[...]
    x2 = _rms_x2(x, ln2_ref[...])
    logits = jax.lax.dot_general(x2, wg_ref[...], (((1,), (1,)), ((), ())),
                                 preferred_element_type=jnp.float32)
    probs = jax.nn.softmax(logits, axis=-1)
    iota = jax.lax.broadcasted_iota(jnp.int32, probs.shape, 1)
    m1 = jnp.max(probs, axis=-1, keepdims=True)
    i1 = jnp.min(jnp.where(probs == m1, iota, E), axis=-1, keepdims=True)
    probs_m = jnp.where(iota == i1, -jnp.inf, probs)
    m2 = jnp.max(probs_m, axis=-1, keepdims=True)
    w0 = m1 / (m1 + m2)
    w1c = m2 / (m1 + m2)

    gu = jax.lax.dot_general(x2.astype(jnp.bfloat16),
                             wse_ref[...].astype(jnp.bfloat16),
                             (((1,), (1,)), ((), ())),
                             preferred_element_type=jnp.float32)
    gate = gu[:, :DFF]
    up = gu[:, DFF:]
    act = (gate * jax.nn.sigmoid(gate) * up).astype(jnp.bfloat16)
    shared = jax.lax.dot_general(act, wsd_ref[...].astype(jnp.bfloat16),
                                 (((1,), (1,)), ((), ())),
                                 preferred_element_type=jnp.float32)
    y0 = y0_ref[0].astype(jnp.float32)
    y1 = y1_ref[0].astype(jnp.float32)
    out_ref[...] = x + w0 * y0 + w1c * y1 + shared


def kernel(hidden_states, positions, kv_cache, attn_metadata, ln1_w, ln2_w,
           Wq, Wkv, Wo, Wg, w1, w2, Wse, Wsd):
    ln2 = ln2_w.reshape(1, HID)

    x2, topi, counts = pl.pallas_call(
        _count_body,
        grid=(T // TBR,),
        in_specs=[
            pl.BlockSpec((TBR, HID), lambda t: (t, 0)),
            pl.BlockSpec((1, HID), lambda t: (0, 0)),
            pl.BlockSpec((E, HID), lambda t: (0, 0)),
        ],
        out_specs=[
            pl.BlockSpec((TBR, HID), lambda t: (t, 0)),
            pl.BlockSpec((2, TBR), lambda t: (0, t)),
            pl.BlockSpec((E, 1), lambda t: (0, 0)),
        ],
        out_shape=[
            jax.ShapeDtypeStruct((T, HID), jnp.bfloat16),
            jax.ShapeDtypeStruct((2, T), jnp.int32),
            jax.ShapeDtypeStruct((E, 1), jnp.float32),
        ],
        compiler_params=pltpu.CompilerParams(
            dimension_semantics=("arbitrary",)),
    )(hidden_states, ln2, Wg)

    pos, te, tv = pl.pallas_call(
        _pos_body,
        grid=(T // TBR,),
        in_specs=[
            pl.BlockSpec((2, TBR), lambda t: (0, t)),
            pl.BlockSpec((E, 1), lambda t: (0, 0)),
        ],
        out_specs=[
            pl.BlockSpec((2, TBR), lambda t: (0, t)),
            pl.BlockSpec((1, NT), lambda t: (0, 0)),
            pl.BlockSpec((1, NT), lambda t: (0, 0)),
        ],
        out_shape=[
            jax.ShapeDtypeStruct((2, T), jnp.int32),
            jax.ShapeDtypeStruct((1, NT), jnp.int32),
            jax.ShapeDtypeStruct((1, NT), jnp.int32),
        ],
        scratch_shapes=[
            pltpu.VMEM((E, 1), jnp.float32),
            pltpu.VMEM((E, 1), jnp.float32),
        ],
        compiler_params=pltpu.CompilerParams(
            dimension_semantics=("arbitrary",)),
    )(topi, counts)

    mesh = plsc.VectorSubcoreMesh(core_axis_name="c", subcore_axis_name="s")

    x2i = jax.lax.bitcast_convert_type(x2.reshape(T, HIDW, 2), jnp.int32)
    scatter_k = functools.partial(
        pl.kernel, mesh=mesh,
        out_type=jax.ShapeDtypeStruct((NTP, HIDW), jnp.int32),
        scratch_types=[
            pltpu.VMEM((CH, HIDW), jnp.int32),
            pltpu.VMEM((CH,), jnp.int32),
            pltpu.VMEM((CH,), jnp.int32),
        ],
    )(_sc_scatter_body)
    xsi = scatter_k(x2i, pos)
    xs = jax.lax.bitcast_convert_type(xsi, jnp.bfloat16).reshape(NTP, HID)

    y = pl.pallas_call(
        _gmm_body,
        grid_spec=pltpu.PrefetchScalarGridSpec(
            num_scalar_prefetch=2,
            grid=(NT,),
            in_specs=[
                pl.BlockSpec((TBM, HID), lambda i, te, tv: (i, 0)),
                pl.BlockSpec((1, DFF, HID), lambda i, te, tv: (te[i], 0, 0)),
                pl.BlockSpec((1, HID, DFF), lambda i, te, tv: (te[i], 0, 0)),
            ],
            out_specs=pl.BlockSpec((TBM, HID), lambda i, te, tv: (i, 0)),
            scratch_shapes=[
                pltpu.VMEM((DFF, HID), jnp.bfloat16),
                pltpu.VMEM((HID, DFF), jnp.bfloat16),
            ],
        ),
        out_shape=jax.ShapeDtypeStruct((NTP, HID), jnp.bfloat16),
        compiler_params=pltpu.CompilerParams(
            dimension_semantics=("arbitrary",)),
    )(te.reshape(NT), tv.reshape(NT), xs, w1, w2)

    yi = jax.lax.bitcast_convert_type(y.reshape(NTP, HIDW, 2), jnp.int32)
    gather_k = functools.partial(
        pl.kernel, mesh=mesh,
        out_type=jax.ShapeDtypeStruct((NSLOT, HIDW), jnp.int32),
        scratch_types=[
            pltpu.VMEM((CH, HIDW), jnp.int32),
            pltpu.VMEM((CH,), jnp.int32),
            pltpu.SemaphoreType.DMA,
        ],
    )(_sc_gather_body)
    ygi = gather_k(yi, pos.reshape(NSLOT))
    yg = jax.lax.bitcast_convert_type(ygi, jnp.bfloat16).reshape(2, T, HID)

    out = pl.pallas_call(
        _combine_body,
        grid=(T // TBC,),
        in_specs=[
            pl.BlockSpec((TBC, HID), lambda t: (t, 0)),
            pl.BlockSpec((1, TBC, HID), lambda t: (0, t, 0)),
            pl.BlockSpec((1, TBC, HID), lambda t: (1, t, 0)),
            pl.BlockSpec((1, HID), lambda t: (0, 0)),
            pl.BlockSpec((E, HID), lambda t: (0, 0)),
            pl.BlockSpec((2 * DFF, HID), lambda t: (0, 0)),
            pl.BlockSpec((HID, DFF), lambda t: (0, 0)),
        ],
        out_specs=pl.BlockSpec((TBC, HID), lambda t: (t, 0)),
        out_shape=jax.ShapeDtypeStruct((T, HID), jnp.float32),
    )(hidden_states, yg, yg, ln2, Wg, Wse, Wsd)

    return out


# speedup vs baseline: 4.4001x; 4.4001x over previous
import functools

import jax
import jax.numpy as jnp
from jax import lax
from jax.experimental import pallas as pl
from jax.experimental.pallas import tpu as pltpu
from jax.experimental.pallas import tpu_sc as plsc

T = 2048
HID = 2048
E = 8
DFF = 768
EPS = 1e-6

TBR = 128
TBM = 256
NSLOT = 2 * T
NT = NSLOT // TBM + E
NTP = NT * TBM
TBC = 256
HIDW = HID // 2

NC = 2
NS = 16
NW = NC * NS
TOK_PER_W = T // NW
ROW_PER_W = NSLOT // NW
CH = 32


def _rms_x2(x, ln2):
    var = jnp.mean(x * x, axis=-1, keepdims=True)
    return x * jax.lax.rsqrt(var + EPS) * ln2


def _count_body(hid_ref, ln2_ref, wg_ref, x2_ref, topi_ref, counts_ref):
    t = pl.program_id(0)
    x = hid_ref[...]
    x2 = _rms_x2(x, ln2_ref[...])
    x2_ref[...] = x2

    logits = jax.lax.dot_general(wg_ref[...], x2, (((1,), (1,)), ((), ())),
                                 preferred_element_type=jnp.float32)
    probs = jax.nn.softmax(logits, axis=0)
    iota_e = jax.lax.broadcasted_iota(jnp.int32, (E, TBR), 0)
    m1 = jnp.max(probs, axis=0, keepdims=True)
    i1 = jnp.min(jnp.where(probs == m1, iota_e, E), axis=0, keepdims=True)
    sel1 = (iota_e == i1).astype(jnp.float32)
    probs_m = jnp.where(iota_e == i1, -jnp.inf, probs)
    m2 = jnp.max(probs_m, axis=0, keepdims=True)
    i2 = jnp.min(jnp.where(probs_m == m2, iota_e, E), axis=0, keepdims=True)
    sel2 = (iota_e == i2).astype(jnp.float32)
    topi_ref[...] = jnp.concatenate([i1, i2], axis=0)

    @pl.when(t == 0)
    def _init_counts():
        counts_ref[...] = jnp.zeros_like(counts_ref)

    counts_ref[...] += (jnp.sum(sel1, axis=1, keepdims=True)
                        + jnp.sum(sel2, axis=1, keepdims=True))


def _pos_body(topi_ref, counts_ref, pos_ref, te_ref, tv_ref,
              run_ref, off_ref):
    t = pl.program_id(0)
    iota_e = jax.lax.broadcasted_iota(jnp.int32, (E, TBR), 0)
    ti = topi_ref[...]
    sel1 = (iota_e == ti[0:1, :]).astype(jnp.float32)
    sel2 = (iota_e == ti[1:2, :]).astype(jnp.float32)

    @pl.when(t == 0)
    def _offsets():
        c = counts_ref[...]
        pc = jnp.floor((c + (TBM - 1)) / TBM) * TBM
        a = jax.lax.broadcasted_iota(jnp.int32, (E, E), 0)
        b = jax.lax.broadcasted_iota(jnp.int32, (E, E), 1)
        lte = (b < a).astype(jnp.float32)
        off = jax.lax.dot_general(lte, pc, (((1,), (0,)), ((), ())),
                                  preferred_element_type=jnp.float32)
        off_ref[...] = off
        run_ref[...] = jnp.zeros_like(run_ref)
        s = (jax.lax.broadcasted_iota(jnp.int32, (E, NT), 1) * TBM
             ).astype(jnp.float32)
        offb = jnp.broadcast_to(off, (E, NT))
        pcb = jnp.broadcast_to(pc, (E, NT))
        mask = ((s >= offb) & (s < offb + pcb)).astype(jnp.int32)
        eio = jax.lax.broadcasted_iota(jnp.int32, (E, NT), 0)
        anym = jnp.max(mask, axis=0, keepdims=True)
        te = jnp.sum(mask * eio, axis=0, keepdims=True)
        te_ref[...] = jnp.where(anym == 1, te, E - 1)
        tv_ref[...] = anym

    run = run_ref[...]
    off = off_ref[...]
    r = jax.lax.broadcasted_iota(jnp.int32, (TBR, TBR), 0)
    c = jax.lax.broadcasted_iota(jnp.int32, (TBR, TBR), 1)
    lt = (r < c).astype(jnp.float32)
    lc1 = jax.lax.dot_general(sel1, lt, (((1,), (0,)), ((), ())),
                              preferred_element_type=jnp.float32)
    pos0 = jnp.sum(sel1 * (off + run + lc1), axis=0, keepdims=True)
    run1 = run + jnp.sum(sel1, axis=1, keepdims=True)
    lc2 = jax.lax.dot_general(sel2, lt, (((1,), (0,)), ((), ())),
                              preferred_element_type=jnp.float32)
    pos1 = jnp.sum(sel2 * (off + run1 + lc2), axis=0, keepdims=True)
    run_ref[...] = run1 + jnp.sum(sel2, axis=1, keepdims=True)
    pos_ref[...] = jnp.concatenate([pos0, pos1], axis=0).astype(jnp.int32)


def _sc_scatter_body(x2_hbm, pos_hbm, xs_hbm, rows_v, idx0_v, idx1_v):
    wid = lax.axis_index("s") * NC + lax.axis_index("c")
    base = wid * TOK_PER_W
    for cch in range(TOK_PER_W // CH):
        tok0 = base + cch * CH
        pltpu.sync_copy(x2_hbm.at[pl.ds(tok0, CH)], rows_v)
        pltpu.sync_copy(pos_hbm.at[0, pl.ds(tok0, CH)], idx0_v)
        pltpu.sync_copy(pos_hbm.at[1, pl.ds(tok0, CH)], idx1_v)
        pltpu.sync_copy(rows_v, xs_hbm.at[idx0_v])
        pltpu.sync_copy(rows_v, xs_hbm.at[idx1_v])


def _gmm_body(te_ref, tv_ref, xs_ref, w1_ref, w2_ref, y_ref,
              w1b_ref, w2b_ref):
    i = pl.program_id(0)
    new_exp = jnp.logical_or(i == 0,
                             te_ref[i] != te_ref[jnp.maximum(i - 1, 0)])

    @pl.when(new_exp)
    def _cache_cast():
        w1b_ref[...] = w1_ref[0].astype(jnp.bfloat16)
        w2b_ref[...] = w2_ref[0].astype(jnp.bfloat16)

    @pl.when(tv_ref[i] == 1)
    def _compute():
        xb = xs_ref[...].astype(jnp.bfloat16)
        h = jax.lax.dot_general(xb, w1b_ref[...],
                                (((1,), (1,)), ((), ())),
                                preferred_element_type=jnp.float32)
        h = (h * jax.nn.sigmoid(h)).astype(jnp.bfloat16)
        y_ref[...] = jax.lax.dot_general(h, w2b_ref[...],
                                         (((1,), (1,)), ((), ())),
                                         preferred_element_type=jnp.float32)


def _sc_gather_body(ys_hbm, posf_hbm, yg_hbm, rows_v, idx_v, sem):
    wid = lax.axis_index("s") * NC + lax.axis_index("c")
    base = wid * ROW_PER_W
    for cch in range(ROW_PER_W // CH):
        r0 = base + cch * CH
        pltpu.sync_copy(posf_hbm.at[pl.ds(r0, CH)], idx_v)
        pltpu.async_copy(ys_hbm.at[idx_v], rows_v, sem).wait()
        pltpu.sync_copy(rows_v, yg_hbm.at[pl.ds(r0, CH)])


def _combine_body(hid_ref, y0_ref, y1_ref, ln2_ref, wg_ref, wse_ref, wsd_ref,
                  out_ref):
    x = hid_ref[...]
    x2 = _rms_x2(x, ln2_ref[...])
    logits = jax.lax.dot_general(x2, wg_ref[...], (((1,), (1,)), ((), ())),
                                 preferred_element_type=jnp.float32)
    probs = jax.nn.softmax(logits, axis=-1)
    iota = jax.lax.broadcasted_iota(jnp.int32, probs.shape, 1)
    m1 = jnp.max(probs, axis=-1, keepdims=True)
    i1 = jnp.min(jnp.where(probs == m1, iota, E), axis=-1, keepdims=True)
    probs_m = jnp.where(iota == i1, -jnp.inf, probs)
    m2 = jnp.max(probs_m, axis=-1, keepdims=True)
    w0 = m1 / (m1 + m2)
    w1c = m2 / (m1 + m2)

    gu = jax.lax.dot_general(x2.astype(jnp.bfloat16),
                             wse_ref[...].astype(jnp.bfloat16),
                             (((1,), (1,)), ((), ())),
                             preferred_element_type=jnp.float32)
    gate = gu[:, :DFF]
    up = gu[:, DFF:]
    act = (gate * jax.nn.sigmoid(gate) * up).astype(jnp.bfloat16)
    shared = jax.lax.dot_general(act, wsd_ref[...].astype(jnp.bfloat16),
                                 (((1,), (1,)), ((), ())),
                                 preferred_element_type=jnp.float32)
    out_ref[...] = x + w0 * y0_ref[0] + w1c * y1_ref[0] + shared


def kernel(hidden_states, positions, kv_cache, attn_metadata, ln1_w, ln2_w,
           Wq, Wkv, Wo, Wg, w1, w2, Wse, Wsd):
    ln2 = ln2_w.reshape(1, HID)

    x2, topi, counts = pl.pallas_call(
        _count_body,
        grid=(T // TBR,),
        in_specs=[
            pl.BlockSpec((TBR, HID), lambda t: (t, 0)),
            pl.BlockSpec((1, HID), lambda t: (0, 0)),
            pl.BlockSpec((E, HID), lambda t: (0, 0)),
        ],
        out_specs=[
            pl.BlockSpec((TBR, HID), lambda t: (t, 0)),
            pl.BlockSpec((2, TBR), lambda t: (0, t)),
            pl.BlockSpec((E, 1), lambda t: (0, 0)),
        ],
        out_shape=[
            jax.ShapeDtypeStruct((T, HID), jnp.float32),
            jax.ShapeDtypeStruct((2, T), jnp.int32),
            jax.ShapeDtypeStruct((E, 1), jnp.float32),
        ],
        compiler_params=pltpu.CompilerParams(
            dimension_semantics=("arbitrary",)),
    )(hidden_states, ln2, Wg)

    pos, te, tv = pl.pallas_call(
        _pos_body,
        grid=(T // TBR,),
        in_specs=[
            pl.BlockSpec((2, TBR), lambda t: (0, t)),
            pl.BlockSpec((E, 1), lambda t: (0, 0)),
        ],
        out_specs=[
            pl.BlockSpec((2, TBR), lambda t: (0, t)),
            pl.BlockSpec((1, NT), lambda t: (0, 0)),
            pl.BlockSpec((1, NT), lambda t: (0, 0)),
        ],
        out_shape=[
            jax.ShapeDtypeStruct((2, T), jnp.int32),
            jax.ShapeDtypeStruct((1, NT), jnp.int32),
            jax.ShapeDtypeStruct((1, NT), jnp.int32),
        ],
        scratch_shapes=[
            pltpu.VMEM((E, 1), jnp.float32),
            pltpu.VMEM((E, 1), jnp.float32),
        ],
        compiler_params=pltpu.CompilerParams(
            dimension_semantics=("arbitrary",)),
    )(topi, counts)

    mesh = plsc.VectorSubcoreMesh(core_axis_name="c", subcore_axis_name="s")

    scatter_k = functools.partial(
        pl.kernel, mesh=mesh,
        out_type=jax.ShapeDtypeStruct((NTP, HID), jnp.float32),
        scratch_types=[
            pltpu.VMEM((CH, HID), jnp.float32),
            pltpu.VMEM((CH,), jnp.int32),
            pltpu.VMEM((CH,), jnp.int32),
        ],
    )(_sc_scatter_body)
    xs = scatter_k(x2, pos)

    y = pl.pallas_call(
        _gmm_body,
        grid_spec=pltpu.PrefetchScalarGridSpec(
            num_scalar_prefetch=2,
            grid=(NT,),
            in_specs=[
                pl.BlockSpec((TBM, HID), lambda i, te, tv: (i, 0)),
                pl.BlockSpec((1, DFF, HID), lambda i, te, tv: (te[i], 0, 0)),
                pl.BlockSpec((1, HID, DFF), lambda i, te, tv: (te[i], 0, 0)),
            ],
            out_specs=pl.BlockSpec((TBM, HID), lambda i, te, tv: (i, 0)),
            scratch_shapes=[
                pltpu.VMEM((DFF, HID), jnp.bfloat16),
                pltpu.VMEM((HID, DFF), jnp.bfloat16),
            ],
        ),
        out_shape=jax.ShapeDtypeStruct((NTP, HID), jnp.float32),
        compiler_params=pltpu.CompilerParams(
            dimension_semantics=("arbitrary",)),
    )(te.reshape(NT), tv.reshape(NT), xs, w1, w2)

    gather_k = functools.partial(
        pl.kernel, mesh=mesh,
        out_type=jax.ShapeDtypeStruct((NSLOT, HID), jnp.float32),
        scratch_types=[
            pltpu.VMEM((CH, HID), jnp.float32),
            pltpu.VMEM((CH,), jnp.int32),
            pltpu.SemaphoreType.DMA,
        ],
    )(_sc_gather_body)
    yg = gather_k(y, pos.reshape(NSLOT)).reshape(2, T, HID)

    out = pl.pallas_call(
        _combine_body,
        grid=(T // TBC,),
        in_specs=[
            pl.BlockSpec((TBC, HID), lambda t: (t, 0)),
            pl.BlockSpec((1, TBC, HID), lambda t: (0, t, 0)),
            pl.BlockSpec((1, TBC, HID), lambda t: (1, t, 0)),
            pl.BlockSpec((1, HID), lambda t: (0, 0)),
            pl.BlockSpec((E, HID), lambda t: (0, 0)),
            pl.BlockSpec((2 * DFF, HID), lambda t: (0, 0)),
            pl.BlockSpec((HID, DFF), lambda t: (0, 0)),
        ],
        out_specs=pl.BlockSpec((TBC, HID), lambda t: (t, 0)),
        out_shape=jax.ShapeDtypeStruct((T, HID), jnp.float32),
    )(hidden_states, yg, yg, ln2, Wg, Wse, Wsd)

    return out


# speedup vs baseline: 5.0729x; 1.1529x over previous
import functools

import jax
import jax.numpy as jnp
from jax import lax
from jax.experimental import pallas as pl
from jax.experimental.pallas import tpu as pltpu
from jax.experimental.pallas import tpu_sc as plsc

T = 2048
HID = 2048
E = 8
DFF = 768
EPS = 1e-6

TBR = 128
TBM = 256
NSLOT = 2 * T
NT = NSLOT // TBM + E
NTP = NT * TBM
TBC = 256
HIDW = HID // 2

NC = 2
NS = 16
NW = NC * NS
TOK_PER_W = T // NW
ROW_PER_W = NSLOT // NW
CH = 32


def _rms_x2(x, ln2):
    var = jnp.mean(x * x, axis=-1, keepdims=True)
    return x * jax.lax.rsqrt(var + EPS) * ln2


def _pack_bf16(x):
    c = x.shape[1] // 2
    lo = jax.lax.bitcast_convert_type(x[:, :c].astype(jnp.bfloat16), jnp.int16)
    hi = jax.lax.bitcast_convert_type(x[:, c:].astype(jnp.bfloat16), jnp.int16)
    lo32 = lo.astype(jnp.int32) & 0xFFFF
    hi32 = hi.astype(jnp.int32) & 0xFFFF
    return (hi32 << 16) | lo32


def _unpack_bf16(w):
    lo = jax.lax.bitcast_convert_type((w & 0xFFFF).astype(jnp.int16),
                                      jnp.bfloat16)
    hi = jax.lax.bitcast_convert_type(
        ((w >> 16) & 0xFFFF).astype(jnp.int16), jnp.bfloat16)
    return jnp.concatenate([lo, hi], axis=1)


def _count_body(hid_ref, ln2_ref, wg_ref, x2_ref, topi_ref, counts_ref):
    t = pl.program_id(0)
    x = hid_ref[...]
    x2 = _rms_x2(x, ln2_ref[...])
    x2_ref[...] = _pack_bf16(x2)

    logits = jax.lax.dot_general(wg_ref[...], x2, (((1,), (1,)), ((), ())),
                                 preferred_element_type=jnp.float32)
    probs = jax.nn.softmax(logits, axis=0)
    iota_e = jax.lax.broadcasted_iota(jnp.int32, (E, TBR), 0)
    m1 = jnp.max(probs, axis=0, keepdims=True)
    i1 = jnp.min(jnp.where(probs == m1, iota_e, E), axis=0, keepdims=True)
    sel1 = (iota_e == i1).astype(jnp.float32)
    probs_m = jnp.where(iota_e == i1, -jnp.inf, probs)
    m2 = jnp.max(probs_m, axis=0, keepdims=True)
    i2 = jnp.min(jnp.where(probs_m == m2, iota_e, E), axis=0, keepdims=True)
    sel2 = (iota_e == i2).astype(jnp.float32)
    topi_ref[...] = jnp.concatenate([i1, i2], axis=0)

    @pl.when(t == 0)
    def _init_counts():
        counts_ref[...] = jnp.zeros_like(counts_ref)

    counts_ref[...] += (jnp.sum(sel1, axis=1, keepdims=True)
                        + jnp.sum(sel2, axis=1, keepdims=True))


def _pos_body(topi_ref, counts_ref, pos_ref, te_ref, tv_ref,
              run_ref, off_ref):
    t = pl.program_id(0)
    iota_e = jax.lax.broadcasted_iota(jnp.int32, (E, TBR), 0)
    ti = topi_ref[...]
    sel1 = (iota_e == ti[0:1, :]).astype(jnp.float32)
    sel2 = (iota_e == ti[1:2, :]).astype(jnp.float32)

    @pl.when(t == 0)
    def _offsets():
        c = counts_ref[...]
        pc = jnp.floor((c + (TBM - 1)) / TBM) * TBM
        a = jax.lax.broadcasted_iota(jnp.int32, (E, E), 0)
        b = jax.lax.broadcasted_iota(jnp.int32, (E, E), 1)
        lte = (b < a).astype(jnp.float32)
        off = jax.lax.dot_general(lte, pc, (((1,), (0,)), ((), ())),
                                  preferred_element_type=jnp.float32)
        off_ref[...] = off
        run_ref[...] = jnp.zeros_like(run_ref)
        s = (jax.lax.broadcasted_iota(jnp.int32, (E, NT), 1) * TBM
             ).astype(jnp.float32)
        offb = jnp.broadcast_to(off, (E, NT))
        pcb = jnp.broadcast_to(pc, (E, NT))
        mask = ((s >= offb) & (s < offb + pcb)).astype(jnp.int32)
        eio = jax.lax.broadcasted_iota(jnp.int32, (E, NT), 0)
        anym = jnp.max(mask, axis=0, keepdims=True)
        te = jnp.sum(mask * eio, axis=0, keepdims=True)
        te_ref[...] = jnp.where(anym == 1, te, E - 1)
        tv_ref[...] = anym

    run = run_ref[...]
    off = off_ref[...]
    r = jax.lax.broadcasted_iota(jnp.int32, (TBR, TBR), 0)
    c = jax.lax.broadcasted_iota(jnp.int32, (TBR, TBR), 1)
    lt = (r < c).astype(jnp.float32)
    lc1 = jax.lax.dot_general(sel1, lt, (((1,), (0,)), ((), ())),
                              preferred_element_type=jnp.float32)
    pos0 = jnp.sum(sel1 * (off + run + lc1), axis=0, keepdims=True)
    run1 = run + jnp.sum(sel1, axis=1, keepdims=True)
    lc2 = jax.lax.dot_general(sel2, lt, (((1,), (0,)), ((), ())),
                              preferred_element_type=jnp.float32)
    pos1 = jnp.sum(sel2 * (off + run1 + lc2), axis=0, keepdims=True)
    run_ref[...] = run1 + jnp.sum(sel2, axis=1, keepdims=True)
    pos_ref[...] = jnp.concatenate([pos0, pos1], axis=0).astype(jnp.int32)


def _sc_scatter_body(x2_hbm, pos_hbm, xs_hbm, rows_v, idx0_v, idx1_v):
    wid = lax.axis_index("s") * NC + lax.axis_index("c")
    base = wid * TOK_PER_W
    for cch in range(TOK_PER_W // CH):
        tok0 = base + cch * CH
        pltpu.sync_copy(x2_hbm.at[pl.ds(tok0, CH)], rows_v)
        pltpu.sync_copy(pos_hbm.at[0, pl.ds(tok0, CH)], idx0_v)
        pltpu.sync_copy(pos_hbm.at[1, pl.ds(tok0, CH)], idx1_v)
        pltpu.sync_copy(rows_v, xs_hbm.at[idx0_v])
        pltpu.sync_copy(rows_v, xs_hbm.at[idx1_v])


def _gmm_body(te_ref, tv_ref, xs_ref, w1_ref, w2_ref, y_ref,
              w1b_ref, w2b_ref):
    i = pl.program_id(0)
    new_exp = jnp.logical_or(i == 0,
                             te_ref[i] != te_ref[jnp.maximum(i - 1, 0)])

    @pl.when(new_exp)
    def _cache_cast():
        w1b_ref[...] = w1_ref[0].astype(jnp.bfloat16)
        w2b_ref[...] = w2_ref[0].astype(jnp.bfloat16)

    @pl.when(tv_ref[i] == 1)
    def _compute():
        xb = _unpack_bf16(xs_ref[...])
        h = jax.lax.dot_general(xb, w1b_ref[...],
                                (((1,), (1,)), ((), ())),
                                preferred_element_type=jnp.float32)
        h = (h * jax.nn.sigmoid(h)).astype(jnp.bfloat16)
        y = jax.lax.dot_general(h, w2b_ref[...],
                                (((1,), (1,)), ((), ())),
                                preferred_element_type=jnp.float32)
        y_ref[...] = _pack_bf16(y)


def _sc_gather_body(ys_hbm, posf_hbm, yg_hbm, rows_v, idx_v, sem):
    wid = lax.axis_index("s") * NC + lax.axis_index("c")
    base = wid * ROW_PER_W
    for cch in range(ROW_PER_W // CH):
        r0 = base + cch * CH
        pltpu.sync_copy(posf_hbm.at[pl.ds(r0, CH)], idx_v)
        pltpu.async_copy(ys_hbm.at[idx_v], rows_v, sem).wait()
        pltpu.sync_copy(rows_v, yg_hbm.at[pl.ds(r0, CH)])


def _combine_body(hid_ref, y0_ref, y1_ref, ln2_ref, wg_ref, wse_ref, wsd_ref,
                  out_ref):
    x = hid_ref[...]
    x2 = _rms_x2(x, ln2_ref[...])
    logits = jax.lax.dot_general(x2, wg_ref[...], (((1,), (1,)), ((), ())),
                                 preferred_element_type=jnp.float32)
    probs = jax.nn.softmax(logits, axis=-1)
    iota = jax.lax.broadcasted_iota(jnp.int32, probs.shape, 1)
    m1 = jnp.max(probs, axis=-1, keepdims=True)
    i1 = jnp.min(jnp.where(probs == m1, iota, E), axis=-1, keepdims=True)
    probs_m = jnp.where(iota == i1, -jnp.inf, probs)
    m2 = jnp.max(probs_m, axis=-1, keepdims=True)
    w0 = m1 / (m1 + m2)
    w1c = m2 / (m1 + m2)

    gu = jax.lax.dot_general(x2.astype(jnp.bfloat16),
                             wse_ref[...].astype(jnp.bfloat16),
                             (((1,), (1,)), ((), ())),
                             preferred_element_type=jnp.float32)
    gate = gu[:, :DFF]
    up = gu[:, DFF:]
    act = (gate * jax.nn.sigmoid(gate) * up).astype(jnp.bfloat16)
    shared = jax.lax.dot_general(act, wsd_ref[...].astype(jnp.bfloat16),
                                 (((1,), (1,)), ((), ())),
                                 preferred_element_type=jnp.float32)
    y0 = _unpack_bf16(y0_ref[0]).astype(jnp.float32)
    y1 = _unpack_bf16(y1_ref[0]).astype(jnp.float32)
    out_ref[...] = x + w0 * y0 + w1c * y1 + shared


def kernel(hidden_states, positions, kv_cache, attn_metadata, ln1_w, ln2_w,
           Wq, Wkv, Wo, Wg, w1, w2, Wse, Wsd):
    ln2 = ln2_w.reshape(1, HID)

    x2, topi, counts = pl.pallas_call(
        _count_body,
        grid=(T // TBR,),
        in_specs=[
            pl.BlockSpec((TBR, HID), lambda t: (t, 0)),
            pl.BlockSpec((1, HID), lambda t: (0, 0)),
            pl.BlockSpec((E, HID), lambda t: (0, 0)),
        ],
        out_specs=[
            pl.BlockSpec((TBR, HIDW), lambda t: (t, 0)),
            pl.BlockSpec((2, TBR), lambda t: (0, t)),
            pl.BlockSpec((E, 1), lambda t: (0, 0)),
        ],
        out_shape=[
            jax.ShapeDtypeStruct((T, HIDW), jnp.int32),
            jax.ShapeDtypeStruct((2, T), jnp.int32),
            jax.ShapeDtypeStruct((E, 1), jnp.float32),
        ],
        compiler_params=pltpu.CompilerParams(
            dimension_semantics=("arbitrary",)),
    )(hidden_states, ln2, Wg)

    pos, te, tv = pl.pallas_call(
        _pos_body,
        grid=(T // TBR,),
        in_specs=[
            pl.BlockSpec((2, TBR), lambda t: (0, t)),
            pl.BlockSpec((E, 1), lambda t: (0, 0)),
        ],
        out_specs=[
            pl.BlockSpec((2, TBR), lambda t: (0, t)),
            pl.BlockSpec((1, NT), lambda t: (0, 0)),
            pl.BlockSpec((1, NT), lambda t: (0, 0)),
        ],
        out_shape=[
            jax.ShapeDtypeStruct((2, T), jnp.int32),
            jax.ShapeDtypeStruct((1, NT), jnp.int32),
            jax.ShapeDtypeStruct((1, NT), jnp.int32),
        ],
        scratch_shapes=[
            pltpu.VMEM((E, 1), jnp.float32),
            pltpu.VMEM((E, 1), jnp.float32),
        ],
        compiler_params=pltpu.CompilerParams(
            dimension_semantics=("arbitrary",)),
    )(topi, counts)

    mesh = plsc.VectorSubcoreMesh(core_axis_name="c", subcore_axis_name="s")

    scatter_k = functools.partial(
        pl.kernel, mesh=mesh,
        out_type=jax.ShapeDtypeStruct((NTP, HIDW), jnp.int32),
        scratch_types=[
            pltpu.VMEM((CH, HIDW), jnp.int32),
            pltpu.VMEM((CH,), jnp.int32),
            pltpu.VMEM((CH,), jnp.int32),
        ],
    )(_sc_scatter_body)
    xs = scatter_k(x2, pos)

    y = pl.pallas_call(
        _gmm_body,
        grid_spec=pltpu.PrefetchScalarGridSpec(
            num_scalar_prefetch=2,
            grid=(NT,),
            in_specs=[
                pl.BlockSpec((TBM, HIDW), lambda i, te, tv: (i, 0)),
                pl.BlockSpec((1, DFF, HID), lambda i, te, tv: (te[i], 0, 0)),
                pl.BlockSpec((1, HID, DFF), lambda i, te, tv: (te[i], 0, 0)),
            ],
            out_specs=pl.BlockSpec((TBM, HIDW), lambda i, te, tv: (i, 0)),
            scratch_shapes=[
                pltpu.VMEM((DFF, HID), jnp.bfloat16),
                pltpu.VMEM((HID, DFF), jnp.bfloat16),
            ],
        ),
        out_shape=jax.ShapeDtypeStruct((NTP, HIDW), jnp.int32),
        compiler_params=pltpu.CompilerParams(
            dimension_semantics=("arbitrary",)),
    )(te.reshape(NT), tv.reshape(NT), xs, w1, w2)

    gather_k = functools.partial(
        pl.kernel, mesh=mesh,
        out_type=jax.ShapeDtypeStruct((NSLOT, HIDW), jnp.int32),
        scratch_types=[
            pltpu.VMEM((CH, HIDW), jnp.int32),
            pltpu.VMEM((CH,), jnp.int32),
            pltpu.SemaphoreType.DMA,
        ],
    )(_sc_gather_body)
    yg = gather_k(y, pos.reshape(NSLOT)).reshape(2, T, HIDW)

    out = pl.pallas_call(
        _combine_body,
        grid=(T // TBC,),
        in_specs=[
            pl.BlockSpec((TBC, HID), lambda t: (t, 0)),
            pl.BlockSpec((1, TBC, HIDW), lambda t: (0, t, 0)),
            pl.BlockSpec((1, TBC, HIDW), lambda t: (1, t, 0)),
            pl.BlockSpec((1, HID), lambda t: (0, 0)),
            pl.BlockSpec((E, HID), lambda t: (0, 0)),
            pl.BlockSpec((2 * DFF, HID), lambda t: (0, 0)),
            pl.BlockSpec((HID, DFF), lambda t: (0, 0)),
        ],
        out_specs=pl.BlockSpec((TBC, HID), lambda t: (t, 0)),
        out_shape=jax.ShapeDtypeStruct((T, HID), jnp.float32),
    )(hidden_states, yg, yg, ln2, Wg, Wse, Wsd)

    return out


# speedup vs baseline: 5.2041x; 1.0259x over previous
import functools

import jax
import jax.numpy as jnp
from jax import lax
from jax.experimental import pallas as pl
from jax.experimental.pallas import tpu as pltpu
from jax.experimental.pallas import tpu_sc as plsc

T = 2048
HID = 2048
E = 8
DFF = 768
EPS = 1e-6

TBR = 128
TBM = 256
NSLOT = 2 * T
NT = NSLOT // TBM + E
NTP = NT * TBM
TBC = 256
HIDW = HID // 2

NC = 2
NS = 16
NW = NC * NS
TOK_PER_W = T // NW
ROW_PER_W = NSLOT // NW
CH = 64


def _rms_x2(x, ln2):
    var = jnp.mean(x * x, axis=-1, keepdims=True)
    return x * jax.lax.rsqrt(var + EPS) * ln2


def _pack_bf16(x):
    c = x.shape[1] // 2
    lo = jax.lax.bitcast_convert_type(x[:, :c].astype(jnp.bfloat16), jnp.int16)
    hi = jax.lax.bitcast_convert_type(x[:, c:].astype(jnp.bfloat16), jnp.int16)
    lo32 = lo.astype(jnp.int32) & 0xFFFF
    hi32 = hi.astype(jnp.int32) & 0xFFFF
    return (hi32 << 16) | lo32


def _unpack_bf16(w):
    lo = jax.lax.bitcast_convert_type((w & 0xFFFF).astype(jnp.int16),
                                      jnp.bfloat16)
    hi = jax.lax.bitcast_convert_type(
        ((w >> 16) & 0xFFFF).astype(jnp.int16), jnp.bfloat16)
    return jnp.concatenate([lo, hi], axis=1)


def _count_body(hid_ref, ln2_ref, wg_ref, x2_ref, topi_ref, counts_ref):
    t = pl.program_id(0)
    x = hid_ref[...]
    x2 = _rms_x2(x, ln2_ref[...])
    x2_ref[...] = _pack_bf16(x2)

    logits = jax.lax.dot_general(wg_ref[...], x2, (((1,), (1,)), ((), ())),
                                 preferred_element_type=jnp.float32)
    probs = jax.nn.softmax(logits, axis=0)
    iota_e = jax.lax.broadcasted_iota(jnp.int32, (E, TBR), 0)
    m1 = jnp.max(probs, axis=0, keepdims=True)
    i1 = jnp.min(jnp.where(probs == m1, iota_e, E), axis=0, keepdims=True)
    sel1 = (iota_e == i1).astype(jnp.float32)
    probs_m = jnp.where(iota_e == i1, -jnp.inf, probs)
    m2 = jnp.max(probs_m, axis=0, keepdims=True)
    i2 = jnp.min(jnp.where(probs_m == m2, iota_e, E), axis=0, keepdims=True)
    sel2 = (iota_e == i2).astype(jnp.float32)
    topi_ref[...] = jnp.concatenate([i1, i2], axis=0)

    @pl.when(t == 0)
    def _init_counts():
        counts_ref[...] = jnp.zeros_like(counts_ref)

    counts_ref[...] += (jnp.sum(sel1, axis=1, keepdims=True)
                        + jnp.sum(sel2, axis=1, keepdims=True))


def _pos_body(topi_ref, counts_ref, pos_ref, te_ref, tv_ref,
              run_ref, off_ref):
    t = pl.program_id(0)
    iota_e = jax.lax.broadcasted_iota(jnp.int32, (E, TBR), 0)
    ti = topi_ref[...]
    sel1 = (iota_e == ti[0:1, :]).astype(jnp.float32)
    sel2 = (iota_e == ti[1:2, :]).astype(jnp.float32)

    @pl.when(t == 0)
    def _offsets():
        c = counts_ref[...]
        pc = jnp.floor((c + (TBM - 1)) / TBM) * TBM
        a = jax.lax.broadcasted_iota(jnp.int32, (E, E), 0)
        b = jax.lax.broadcasted_iota(jnp.int32, (E, E), 1)
        lte = (b < a).astype(jnp.float32)
        off = jax.lax.dot_general(lte, pc, (((1,), (0,)), ((), ())),
                                  preferred_element_type=jnp.float32)
        off_ref[...] = off
        run_ref[...] = jnp.zeros_like(run_ref)
        s = (jax.lax.broadcasted_iota(jnp.int32, (E, NT), 1) * TBM
             ).astype(jnp.float32)
        offb = jnp.broadcast_to(off, (E, NT))
        pcb = jnp.broadcast_to(pc, (E, NT))
        mask = ((s >= offb) & (s < offb + pcb)).astype(jnp.int32)
        eio = jax.lax.broadcasted_iota(jnp.int32, (E, NT), 0)
        anym = jnp.max(mask, axis=0, keepdims=True)
        te = jnp.sum(mask * eio, axis=0, keepdims=True)
        te_ref[...] = jnp.where(anym == 1, te, E - 1)
        tv_ref[...] = anym

    run = run_ref[...]
    off = off_ref[...]
    r = jax.lax.broadcasted_iota(jnp.int32, (TBR, TBR), 0)
    c = jax.lax.broadcasted_iota(jnp.int32, (TBR, TBR), 1)
    lt = (r < c).astype(jnp.float32)
    lc1 = jax.lax.dot_general(sel1, lt, (((1,), (0,)), ((), ())),
                              preferred_element_type=jnp.float32)
    pos0 = jnp.sum(sel1 * (off + run + lc1), axis=0, keepdims=True)
    run1 = run + jnp.sum(sel1, axis=1, keepdims=True)
    lc2 = jax.lax.dot_general(sel2, lt, (((1,), (0,)), ((), ())),
                              preferred_element_type=jnp.float32)
    pos1 = jnp.sum(sel2 * (off + run1 + lc2), axis=0, keepdims=True)
    run_ref[...] = run1 + jnp.sum(sel2, axis=1, keepdims=True)
    pos_ref[...] = jnp.concatenate([pos0, pos1], axis=0).astype(jnp.int32)


def _sc_scatter_body(x2_hbm, pos_hbm, xs_hbm, rows_v, idx0_v, idx1_v):
    wid = lax.axis_index("s") * NC + lax.axis_index("c")
    base = wid * TOK_PER_W
    for cch in range(TOK_PER_W // CH):
        tok0 = base + cch * CH
        pltpu.sync_copy(x2_hbm.at[pl.ds(tok0, CH)], rows_v)
        pltpu.sync_copy(pos_hbm.at[0, pl.ds(tok0, CH)], idx0_v)
        pltpu.sync_copy(pos_hbm.at[1, pl.ds(tok0, CH)], idx1_v)
        pltpu.sync_copy(rows_v, xs_hbm.at[idx0_v])
        pltpu.sync_copy(rows_v, xs_hbm.at[idx1_v])


def _gmm_body(te_ref, tv_ref, xs_ref, w1_ref, w2_ref, y_ref,
              w1b_ref, w2b_ref):
    i = pl.program_id(0)
    new_exp = jnp.logical_or(i == 0,
                             te_ref[i] != te_ref[jnp.maximum(i - 1, 0)])

    @pl.when(new_exp)
    def _cache_cast():
        w1b_ref[...] = w1_ref[0].astype(jnp.bfloat16)
        w2b_ref[...] = w2_ref[0].astype(jnp.bfloat16)

    @pl.when(tv_ref[i] == 1)
    def _compute():
        xb = _unpack_bf16(xs_ref[...])
        h = jax.lax.dot_general(xb, w1b_ref[...],
                                (((1,), (1,)), ((), ())),
                                preferred_element_type=jnp.float32)
        h = (h * jax.nn.sigmoid(h)).astype(jnp.bfloat16)
        y = jax.lax.dot_general(h, w2b_ref[...],
                                (((1,), (1,)), ((), ())),
                                preferred_element_type=jnp.float32)
        y_ref[...] = _pack_bf16(y)


def _sc_gather_body(ys_hbm, posf_hbm, yg_hbm, rows_v, idx_v, sem):
    wid = lax.axis_index("s") * NC + lax.axis_index("c")
    base = wid * ROW_PER_W
    for cch in range(ROW_PER_W // CH):
        r0 = base + cch * CH
        pltpu.sync_copy(posf_hbm.at[pl.ds(r0, CH)], idx_v)
        pltpu.async_copy(ys_hbm.at[idx_v], rows_v, sem).wait()
        pltpu.sync_copy(rows_v, yg_hbm.at[pl.ds(r0, CH)])


def _combine_body(hid_ref, y0_ref, y1_ref, ln2_ref, wg_ref, wse_ref, wsd_ref,
                  out_ref):
    x = hid_ref[...]
    x2 = _rms_x2(x, ln2_ref[...])
    logits = jax.lax.dot_general(x2, wg_ref[...], (((1,), (1,)), ((), ())),
                                 preferred_element_type=jnp.float32)
    probs = jax.nn.softmax(logits, axis=-1)
    iota = jax.lax.broadcasted_iota(jnp.int32, probs.shape, 1)
    m1 = jnp.max(probs, axis=-1, keepdims=True)
    i1 = jnp.min(jnp.where(probs == m1, iota, E), axis=-1, keepdims=True)
    probs_m = jnp.where(iota == i1, -jnp.inf, probs)
    m2 = jnp.max(probs_m, axis=-1, keepdims=True)
    w0 = m1 / (m1 + m2)
    w1c = m2 / (m1 + m2)

    gu = jax.lax.dot_general(x2.astype(jnp.bfloat16),
                             wse_ref[...].astype(jnp.bfloat16),
                             (((1,), (1,)), ((), ())),
                             preferred_element_type=jnp.float32)
    gate = gu[:, :DFF]
    up = gu[:, DFF:]
    act = (gate * jax.nn.sigmoid(gate) * up).astype(jnp.bfloat16)
    shared = jax.lax.dot_general(act, wsd_ref[...].astype(jnp.bfloat16),
                                 (((1,), (1,)), ((), ())),
                                 preferred_element_type=jnp.float32)
    y0 = _unpack_bf16(y0_ref[0]).astype(jnp.float32)
    y1 = _unpack_bf16(y1_ref[0]).astype(jnp.float32)
    out_ref[...] = x + w0 * y0 + w1c * y1 + shared


def kernel(hidden_states, positions, kv_cache, attn_metadata, ln1_w, ln2_w,
           Wq, Wkv, Wo, Wg, w1, w2, Wse, Wsd):
    ln2 = ln2_w.reshape(1, HID)

    x2, topi, counts = pl.pallas_call(
        _count_body,
        grid=(T // TBR,),
        in_specs=[
            pl.BlockSpec((TBR, HID), lambda t: (t, 0)),
            pl.BlockSpec((1, HID), lambda t: (0, 0)),
            pl.BlockSpec((E, HID), lambda t: (0, 0)),
        ],
        out_specs=[
            pl.BlockSpec((TBR, HIDW), lambda t: (t, 0)),
            pl.BlockSpec((2, TBR), lambda t: (0, t)),
            pl.BlockSpec((E, 1), lambda t: (0, 0)),
        ],
        out_shape=[
            jax.ShapeDtypeStruct((T, HIDW), jnp.int32),
            jax.ShapeDtypeStruct((2, T), jnp.int32),
            jax.ShapeDtypeStruct((E, 1), jnp.float32),
        ],
        compiler_params=pltpu.CompilerParams(
            dimension_semantics=("arbitrary",)),
    )(hidden_states, ln2, Wg)

    pos, te, tv = pl.pallas_call(
        _pos_body,
        grid=(T // TBR,),
        in_specs=[
            pl.BlockSpec((2, TBR), lambda t: (0, t)),
            pl.BlockSpec((E, 1), lambda t: (0, 0)),
        ],
        out_specs=[
            pl.BlockSpec((2, TBR), lambda t: (0, t)),
            pl.BlockSpec((1, NT), lambda t: (0, 0)),
            pl.BlockSpec((1, NT), lambda t: (0, 0)),
        ],
        out_shape=[
            jax.ShapeDtypeStruct((2, T), jnp.int32),
            jax.ShapeDtypeStruct((1, NT), jnp.int32),
            jax.ShapeDtypeStruct((1, NT), jnp.int32),
        ],
        scratch_shapes=[
            pltpu.VMEM((E, 1), jnp.float32),
            pltpu.VMEM((E, 1), jnp.float32),
        ],
        compiler_params=pltpu.CompilerParams(
            dimension_semantics=("arbitrary",)),
    )(topi, counts)

    mesh = plsc.VectorSubcoreMesh(core_axis_name="c", subcore_axis_name="s")

    scatter_k = functools.partial(
        pl.kernel, mesh=mesh,
        out_type=jax.ShapeDtypeStruct((NTP, HIDW), jnp.int32),
        scratch_types=[
            pltpu.VMEM((CH, HIDW), jnp.int32),
            pltpu.VMEM((CH,), jnp.int32),
            pltpu.VMEM((CH,), jnp.int32),
        ],
    )(_sc_scatter_body)
    xs = scatter_k(x2, pos)

    y = pl.pallas_call(
        _gmm_body,
        grid_spec=pltpu.PrefetchScalarGridSpec(
            num_scalar_prefetch=2,
            grid=(NT,),
            in_specs=[
                pl.BlockSpec((TBM, HIDW), lambda i, te, tv: (i, 0)),
                pl.BlockSpec((1, DFF, HID), lambda i, te, tv: (te[i], 0, 0)),
                pl.BlockSpec((1, HID, DFF), lambda i, te, tv: (te[i], 0, 0)),
            ],
            out_specs=pl.BlockSpec((TBM, HIDW), lambda i, te, tv: (i, 0)),
            scratch_shapes=[
                pltpu.VMEM((DFF, HID), jnp.bfloat16),
                pltpu.VMEM((HID, DFF), jnp.bfloat16),
            ],
        ),
        out_shape=jax.ShapeDtypeStruct((NTP, HIDW), jnp.int32),
        compiler_params=pltpu.CompilerParams(
            dimension_semantics=("arbitrary",)),
    )(te.reshape(NT), tv.reshape(NT), xs, w1, w2)

    gather_k = functools.partial(
        pl.kernel, mesh=mesh,
        out_type=jax.ShapeDtypeStruct((NSLOT, HIDW), jnp.int32),
        scratch_types=[
            pltpu.VMEM((CH, HIDW), jnp.int32),
            pltpu.VMEM((CH,), jnp.int32),
            pltpu.SemaphoreType.DMA,
        ],
    )(_sc_gather_body)
    yg = gather_k(y, pos.reshape(NSLOT)).reshape(2, T, HIDW)

    out = pl.pallas_call(
        _combine_body,
        grid=(T // TBC,),
        in_specs=[
            pl.BlockSpec((TBC, HID), lambda t: (t, 0)),
            pl.BlockSpec((1, TBC, HIDW), lambda t: (0, t, 0)),
            pl.BlockSpec((1, TBC, HIDW), lambda t: (1, t, 0)),
            pl.BlockSpec((1, HID), lambda t: (0, 0)),
            pl.BlockSpec((E, HID), lambda t: (0, 0)),
            pl.BlockSpec((2 * DFF, HID), lambda t: (0, 0)),
            pl.BlockSpec((HID, DFF), lambda t: (0, 0)),
        ],
        out_specs=pl.BlockSpec((TBC, HID), lambda t: (t, 0)),
        out_shape=jax.ShapeDtypeStruct((T, HID), jnp.float32),
    )(hidden_states, yg, yg, ln2, Wg, Wse, Wsd)

    return out


# speedup vs baseline: 5.4494x; 1.0471x over previous
import functools

import jax
import jax.numpy as jnp
from jax import lax
from jax.experimental import pallas as pl
from jax.experimental.pallas import tpu as pltpu
from jax.experimental.pallas import tpu_sc as plsc

T = 2048
HID = 2048
E = 8
DFF = 768
EPS = 1e-6

TBR = 256
TBM = 256
NSLOT = 2 * T
NT = NSLOT // TBM + E
NTP = NT * TBM
TBC = 512
HIDW = HID // 2

NC = 2
NS = 16
NW = NC * NS
TOK_PER_W = T // NW
ROW_PER_W = NSLOT // NW
CH = 64


def _rms_x2(x, ln2):
    var = jnp.mean(x * x, axis=-1, keepdims=True)
    return x * jax.lax.rsqrt(var + EPS) * ln2


def _pack_bf16(x):
    c = x.shape[1] // 2
    lo = jax.lax.bitcast_convert_type(x[:, :c].astype(jnp.bfloat16), jnp.int16)
    hi = jax.lax.bitcast_convert_type(x[:, c:].astype(jnp.bfloat16), jnp.int16)
    lo32 = lo.astype(jnp.int32) & 0xFFFF
    hi32 = hi.astype(jnp.int32) & 0xFFFF
    return (hi32 << 16) | lo32


def _unpack_bf16(w):
    lo = jax.lax.bitcast_convert_type((w & 0xFFFF).astype(jnp.int16),
                                      jnp.bfloat16)
    hi = jax.lax.bitcast_convert_type(
        ((w >> 16) & 0xFFFF).astype(jnp.int16), jnp.bfloat16)
    return jnp.concatenate([lo, hi], axis=1)


def _count_body(hid_ref, ln2_ref, wg_ref, x2_ref, topi_ref, counts_ref):
    t = pl.program_id(0)
    x = hid_ref[...]
    x2 = _rms_x2(x, ln2_ref[...])
    x2_ref[...] = _pack_bf16(x2)

    logits = jax.lax.dot_general(wg_ref[...], x2, (((1,), (1,)), ((), ())),
                                 preferred_element_type=jnp.float32)
    probs = jax.nn.softmax(logits, axis=0)
    iota_e = jax.lax.broadcasted_iota(jnp.int32, (E, TBR), 0)
    m1 = jnp.max(probs, axis=0, keepdims=True)
    i1 = jnp.min(jnp.where(probs == m1, iota_e, E), axis=0, keepdims=True)
    sel1 = (iota_e == i1).astype(jnp.float32)
    probs_m = jnp.where(iota_e == i1, -jnp.inf, probs)
    m2 = jnp.max(probs_m, axis=0, keepdims=True)
    i2 = jnp.min(jnp.where(probs_m == m2, iota_e, E), axis=0, keepdims=True)
    sel2 = (iota_e == i2).astype(jnp.float32)
    topi_ref[...] = jnp.concatenate([i1, i2], axis=0)

    @pl.when(t == 0)
    def _init_counts():
        counts_ref[...] = jnp.zeros_like(counts_ref)

    counts_ref[...] += (jnp.sum(sel1, axis=1, keepdims=True)
                        + jnp.sum(sel2, axis=1, keepdims=True))


def _pos_body(topi_ref, counts_ref, pos_ref, te_ref, tv_ref,
              run_ref, off_ref):
    t = pl.program_id(0)
    iota_e = jax.lax.broadcasted_iota(jnp.int32, (E, TBR), 0)
    ti = topi_ref[...]
    sel1 = (iota_e == ti[0:1, :]).astype(jnp.float32)
    sel2 = (iota_e == ti[1:2, :]).astype(jnp.float32)

    @pl.when(t == 0)
    def _offsets():
        c = counts_ref[...]
        pc = jnp.floor((c + (TBM - 1)) / TBM) * TBM
        a = jax.lax.broadcasted_iota(jnp.int32, (E, E), 0)
        b = jax.lax.broadcasted_iota(jnp.int32, (E, E), 1)
        lte = (b < a).astype(jnp.float32)
        off = jax.lax.dot_general(lte, pc, (((1,), (0,)), ((), ())),
                                  preferred_element_type=jnp.float32)
        off_ref[...] = off
        run_ref[...] = jnp.zeros_like(run_ref)
        s = (jax.lax.broadcasted_iota(jnp.int32, (E, NT), 1) * TBM
             ).astype(jnp.float32)
        offb = jnp.broadcast_to(off, (E, NT))
        pcb = jnp.broadcast_to(pc, (E, NT))
        mask = ((s >= offb) & (s < offb + pcb)).astype(jnp.int32)
        eio = jax.lax.broadcasted_iota(jnp.int32, (E, NT), 0)
        anym = jnp.max(mask, axis=0, keepdims=True)
        te = jnp.sum(mask * eio, axis=0, keepdims=True)
        te_ref[...] = jnp.where(anym == 1, te, E - 1)
        tv_ref[...] = anym

    run = run_ref[...]
    off = off_ref[...]
    r = jax.lax.broadcasted_iota(jnp.int32, (TBR, TBR), 0)
    c = jax.lax.broadcasted_iota(jnp.int32, (TBR, TBR), 1)
    lt = (r < c).astype(jnp.float32)
    lc1 = jax.lax.dot_general(sel1, lt, (((1,), (0,)), ((), ())),
                              preferred_element_type=jnp.float32)
    pos0 = jnp.sum(sel1 * (off + run + lc1), axis=0, keepdims=True)
    run1 = run + jnp.sum(sel1, axis=1, keepdims=True)
    lc2 = jax.lax.dot_general(sel2, lt, (((1,), (0,)), ((), ())),
                              preferred_element_type=jnp.float32)
    pos1 = jnp.sum(sel2 * (off + run1 + lc2), axis=0, keepdims=True)
    run_ref[...] = run1 + jnp.sum(sel2, axis=1, keepdims=True)
    pos_ref[...] = jnp.concatenate([pos0, pos1], axis=0).astype(jnp.int32)


def _sc_scatter_body(x2_hbm, pos_hbm, xs_hbm, rows_v, idx0_v, idx1_v):
    wid = lax.axis_index("s") * NC + lax.axis_index("c")
    base = wid * TOK_PER_W
    for cch in range(TOK_PER_W // CH):
        tok0 = base + cch * CH
        pltpu.sync_copy(x2_hbm.at[pl.ds(tok0, CH)], rows_v)
        pltpu.sync_copy(pos_hbm.at[0, pl.ds(tok0, CH)], idx0_v)
        pltpu.sync_copy(pos_hbm.at[1, pl.ds(tok0, CH)], idx1_v)
        pltpu.sync_copy(rows_v, xs_hbm.at[idx0_v])
        pltpu.sync_copy(rows_v, xs_hbm.at[idx1_v])


def _gmm_body(te_ref, tv_ref, xs_ref, w1_ref, w2_ref, y_ref,
              w1b_ref, w2b_ref):
    i = pl.program_id(0)
    new_exp = jnp.logical_or(i == 0,
                             te_ref[i] != te_ref[jnp.maximum(i - 1, 0)])

    @pl.when(new_exp)
    def _cache_cast():
        w1b_ref[...] = w1_ref[0].astype(jnp.bfloat16)
        w2b_ref[...] = w2_ref[0].astype(jnp.bfloat16)

    @pl.when(tv_ref[i] == 1)
    def _compute():
        xb = _unpack_bf16(xs_ref[...])
        h = jax.lax.dot_general(xb, w1b_ref[...],
                                (((1,), (1,)), ((), ())),
                                preferred_element_type=jnp.float32)
        h = (h * jax.nn.sigmoid(h)).astype(jnp.bfloat16)
        y = jax.lax.dot_general(h, w2b_ref[...],
                                (((1,), (1,)), ((), ())),
                                preferred_element_type=jnp.float32)
        y_ref[...] = _pack_bf16(y)


def _sc_gather_body(ys_hbm, posf_hbm, yg_hbm, rows_v, idx_v, sem):
    wid = lax.axis_index("s") * NC + lax.axis_index("c")
    base = wid * ROW_PER_W
    for cch in range(ROW_PER_W // CH):
        r0 = base + cch * CH
        pltpu.sync_copy(posf_hbm.at[pl.ds(r0, CH)], idx_v)
        pltpu.async_copy(ys_hbm.at[idx_v], rows_v, sem).wait()
        pltpu.sync_copy(rows_v, yg_hbm.at[pl.ds(r0, CH)])


def _combine_body(hid_ref, y0_ref, y1_ref, ln2_ref, wg_ref, wse_ref, wsd_ref,
                  out_ref):
    x = hid_ref[...]
    x2 = _rms_x2(x, ln2_ref[...])
    logits = jax.lax.dot_general(x2, wg_ref[...], (((1,), (1,)), ((), ())),
                                 preferred_element_type=jnp.float32)
    probs = jax.nn.softmax(logits, axis=-1)
    iota = jax.lax.broadcasted_iota(jnp.int32, probs.shape, 1)
    m1 = jnp.max(probs, axis=-1, keepdims=True)
    i1 = jnp.min(jnp.where(probs == m1, iota, E), axis=-1, keepdims=True)
    probs_m = jnp.where(iota == i1, -jnp.inf, probs)
    m2 = jnp.max(probs_m, axis=-1, keepdims=True)
    w0 = m1 / (m1 + m2)
    w1c = m2 / (m1 + m2)

    gu = jax.lax.dot_general(x2.astype(jnp.bfloat16),
                             wse_ref[...].astype(jnp.bfloat16),
                             (((1,), (1,)), ((), ())),
                             preferred_element_type=jnp.float32)
    gate = gu[:, :DFF]
    up = gu[:, DFF:]
    act = (gate * jax.nn.sigmoid(gate) * up).astype(jnp.bfloat16)
    shared = jax.lax.dot_general(act, wsd_ref[...].astype(jnp.bfloat16),
                                 (((1,), (1,)), ((), ())),
                                 preferred_element_type=jnp.float32)
    y0 = _unpack_bf16(y0_ref[0]).astype(jnp.float32)
    y1 = _unpack_bf16(y1_ref[0]).astype(jnp.float32)
    out_ref[...] = x + w0 * y0 + w1c * y1 + shared


def kernel(hidden_states, positions, kv_cache, attn_metadata, ln1_w, ln2_w,
           Wq, Wkv, Wo, Wg, w1, w2, Wse, Wsd):
    ln2 = ln2_w.reshape(1, HID)

    x2, topi, counts = pl.pallas_call(
        _count_body,
        grid=(T // TBR,),
        in_specs=[
            pl.BlockSpec((TBR, HID), lambda t: (t, 0)),
            pl.BlockSpec((1, HID), lambda t: (0, 0)),
            pl.BlockSpec((E, HID), lambda t: (0, 0)),
        ],
        out_specs=[
            pl.BlockSpec((TBR, HIDW), lambda t: (t, 0)),
            pl.BlockSpec((2, TBR), lambda t: (0, t)),
            pl.BlockSpec((E, 1), lambda t: (0, 0)),
        ],
        out_shape=[
            jax.ShapeDtypeStruct((T, HIDW), jnp.int32),
            jax.ShapeDtypeStruct((2, T), jnp.int32),
            jax.ShapeDtypeStruct((E, 1), jnp.float32),
        ],
        compiler_params=pltpu.CompilerParams(
            dimension_semantics=("arbitrary",)),
    )(hidden_states, ln2, Wg)

    pos, te, tv = pl.pallas_call(
        _pos_body,
        grid=(T // TBR,),
        in_specs=[
            pl.BlockSpec((2, TBR), lambda t: (0, t)),
            pl.BlockSpec((E, 1), lambda t: (0, 0)),
        ],
        out_specs=[
            pl.BlockSpec((2, TBR), lambda t: (0, t)),
            pl.BlockSpec((1, NT), lambda t: (0, 0)),
            pl.BlockSpec((1, NT), lambda t: (0, 0)),
        ],
        out_shape=[
            jax.ShapeDtypeStruct((2, T), jnp.int32),
            jax.ShapeDtypeStruct((1, NT), jnp.int32),
            jax.ShapeDtypeStruct((1, NT), jnp.int32),
        ],
        scratch_shapes=[
            pltpu.VMEM((E, 1), jnp.float32),
            pltpu.VMEM((E, 1), jnp.float32),
        ],
        compiler_params=pltpu.CompilerParams(
            dimension_semantics=("arbitrary",)),
    )(topi, counts)

    mesh = plsc.VectorSubcoreMesh(core_axis_name="c", subcore_axis_name="s")

    scatter_k = functools.partial(
        pl.kernel, mesh=mesh,
        out_type=jax.ShapeDtypeStruct((NTP, HIDW), jnp.int32),
        scratch_types=[
            pltpu.VMEM((CH, HIDW), jnp.int32),
            pltpu.VMEM((CH,), jnp.int32),
            pltpu.VMEM((CH,), jnp.int32),
        ],
    )(_sc_scatter_body)
    xs = scatter_k(x2, pos)

    y = pl.pallas_call(
        _gmm_body,
        grid_spec=pltpu.PrefetchScalarGridSpec(
            num_scalar_prefetch=2,
            grid=(NT,),
            in_specs=[
                pl.BlockSpec((TBM, HIDW), lambda i, te, tv: (i, 0)),
                pl.BlockSpec((1, DFF, HID), lambda i, te, tv: (te[i], 0, 0)),
                pl.BlockSpec((1, HID, DFF), lambda i, te, tv: (te[i], 0, 0)),
            ],
            out_specs=pl.BlockSpec((TBM, HIDW), lambda i, te, tv: (i, 0)),
            scratch_shapes=[
                pltpu.VMEM((DFF, HID), jnp.bfloat16),
                pltpu.VMEM((HID, DFF), jnp.bfloat16),
            ],
        ),
        out_shape=jax.ShapeDtypeStruct((NTP, HIDW), jnp.int32),
        compiler_params=pltpu.CompilerParams(
            dimension_semantics=("arbitrary",)),
    )(te.reshape(NT), tv.reshape(NT), xs, w1, w2)

    gather_k = functools.partial(
        pl.kernel, mesh=mesh,
        out_type=jax.ShapeDtypeStruct((NSLOT, HIDW), jnp.int32),
        scratch_types=[
            pltpu.VMEM((CH, HIDW), jnp.int32),
            pltpu.VMEM((CH,), jnp.int32),
            pltpu.SemaphoreType.DMA,
        ],
    )(_sc_gather_body)
    yg = gather_k(y, pos.reshape(NSLOT)).reshape(2, T, HIDW)

    out = pl.pallas_call(
        _combine_body,
        grid=(T // TBC,),
        in_specs=[
            pl.BlockSpec((TBC, HID), lambda t: (t, 0)),
            pl.BlockSpec((1, TBC, HIDW), lambda t: (0, t, 0)),
            pl.BlockSpec((1, TBC, HIDW), lambda t: (1, t, 0)),
            pl.BlockSpec((1, HID), lambda t: (0, 0)),
            pl.BlockSpec((E, HID), lambda t: (0, 0)),
            pl.BlockSpec((2 * DFF, HID), lambda t: (0, 0)),
            pl.BlockSpec((HID, DFF), lambda t: (0, 0)),
        ],
        out_specs=pl.BlockSpec((TBC, HID), lambda t: (t, 0)),
        out_shape=jax.ShapeDtypeStruct((T, HID), jnp.float32),
    )(hidden_states, yg, yg, ln2, Wg, Wse, Wsd)

    return out


# speedup vs baseline: 5.5484x; 1.0182x over previous
import functools

import jax
import jax.numpy as jnp
from jax import lax
from jax.experimental import pallas as pl
from jax.experimental.pallas import tpu as pltpu
from jax.experimental.pallas import tpu_sc as plsc

T = 2048
HID = 2048
E = 8
DFF = 768
EPS = 1e-6

TBR = 512
TBM = 256
NSLOT = 2 * T
NT = NSLOT // TBM + E
NTP = NT * TBM
TBC = 512
HIDW = HID // 2

NC = 2
NS = 16
NW = NC * NS
TOK_PER_W = T // NW
ROW_PER_W = NSLOT // NW
CH = 64


def _rms_x2(x, ln2):
    var = jnp.mean(x * x, axis=-1, keepdims=True)
    return x * jax.lax.rsqrt(var + EPS) * ln2


def _pack_bf16(x):
    c = x.shape[1] // 2
    lo = jax.lax.bitcast_convert_type(x[:, :c].astype(jnp.bfloat16), jnp.int16)
    hi = jax.lax.bitcast_convert_type(x[:, c:].astype(jnp.bfloat16), jnp.int16)
    lo32 = lo.astype(jnp.int32) & 0xFFFF
    hi32 = hi.astype(jnp.int32) & 0xFFFF
    return (hi32 << 16) | lo32


def _unpack_bf16(w):
    lo = jax.lax.bitcast_convert_type((w & 0xFFFF).astype(jnp.int16),
                                      jnp.bfloat16)
    hi = jax.lax.bitcast_convert_type(
        ((w >> 16) & 0xFFFF).astype(jnp.int16), jnp.bfloat16)
    return jnp.concatenate([lo, hi], axis=1)


def _count_body(hid_ref, ln2_ref, wg_ref, x2_ref, topi_ref, counts_ref):
    t = pl.program_id(0)
    x = hid_ref[...]
    x2 = _rms_x2(x, ln2_ref[...])
    x2_ref[...] = _pack_bf16(x2)

    logits = jax.lax.dot_general(wg_ref[...], x2, (((1,), (1,)), ((), ())),
                                 preferred_element_type=jnp.float32)
    probs = jax.nn.softmax(logits, axis=0)
    iota_e = jax.lax.broadcasted_iota(jnp.int32, (E, TBR), 0)
    m1 = jnp.max(probs, axis=0, keepdims=True)
    i1 = jnp.min(jnp.where(probs == m1, iota_e, E), axis=0, keepdims=True)
    sel1 = (iota_e == i1).astype(jnp.float32)
    probs_m = jnp.where(iota_e == i1, -jnp.inf, probs)
    m2 = jnp.max(probs_m, axis=0, keepdims=True)
    i2 = jnp.min(jnp.where(probs_m == m2, iota_e, E), axis=0, keepdims=True)
    sel2 = (iota_e == i2).astype(jnp.float32)
    topi_ref[...] = jnp.concatenate([i1, i2], axis=0)

    @pl.when(t == 0)
    def _init_counts():
        counts_ref[...] = jnp.zeros_like(counts_ref)

    counts_ref[...] += (jnp.sum(sel1, axis=1, keepdims=True)
                        + jnp.sum(sel2, axis=1, keepdims=True))


def _pos_body(topi_ref, counts_ref, pos_ref, te_ref, tv_ref,
              run_ref, off_ref):
    t = pl.program_id(0)
    iota_e = jax.lax.broadcasted_iota(jnp.int32, (E, TBR), 0)
    ti = topi_ref[...]
    sel1 = (iota_e == ti[0:1, :]).astype(jnp.float32)
    sel2 = (iota_e == ti[1:2, :]).astype(jnp.float32)

    @pl.when(t == 0)
    def _offsets():
        c = counts_ref[...]
        pc = jnp.floor((c + (TBM - 1)) / TBM) * TBM
        a = jax.lax.broadcasted_iota(jnp.int32, (E, E), 0)
        b = jax.lax.broadcasted_iota(jnp.int32, (E, E), 1)
        lte = (b < a).astype(jnp.float32)
        off = jax.lax.dot_general(lte, pc, (((1,), (0,)), ((), ())),
                                  preferred_element_type=jnp.float32)
        off_ref[...] = off
        run_ref[...] = jnp.zeros_like(run_ref)
        s = (jax.lax.broadcasted_iota(jnp.int32, (E, NT), 1) * TBM
             ).astype(jnp.float32)
        offb = jnp.broadcast_to(off, (E, NT))
        pcb = jnp.broadcast_to(pc, (E, NT))
        mask = ((s >= offb) & (s < offb + pcb)).astype(jnp.int32)
        eio = jax.lax.broadcasted_iota(jnp.int32, (E, NT), 0)
        anym = jnp.max(mask, axis=0, keepdims=True)
        te = jnp.sum(mask * eio, axis=0, keepdims=True)
        te_ref[...] = jnp.where(anym == 1, te, E - 1)
        tv_ref[...] = anym

    run = run_ref[...]
    off = off_ref[...]
    r = jax.lax.broadcasted_iota(jnp.int32, (TBR, TBR), 0)
    c = jax.lax.broadcasted_iota(jnp.int32, (TBR, TBR), 1)
    lt = (r < c).astype(jnp.float32)
    lc1 = jax.lax.dot_general(sel1, lt, (((1,), (0,)), ((), ())),
                              preferred_element_type=jnp.float32)
    pos0 = jnp.sum(sel1 * (off + run + lc1), axis=0, keepdims=True)
    run1 = run + jnp.sum(sel1, axis=1, keepdims=True)
    lc2 = jax.lax.dot_general(sel2, lt, (((1,), (0,)), ((), ())),
                              preferred_element_type=jnp.float32)
    pos1 = jnp.sum(sel2 * (off + run1 + lc2), axis=0, keepdims=True)
    run_ref[...] = run1 + jnp.sum(sel2, axis=1, keepdims=True)
    pos_ref[...] = jnp.concatenate([pos0, pos1], axis=0).astype(jnp.int32)


def _sc_scatter_body(x2_hbm, pos_hbm, xs_hbm, rows_v, idx0_v, idx1_v):
    wid = lax.axis_index("s") * NC + lax.axis_index("c")
    base = wid * TOK_PER_W
    for cch in range(TOK_PER_W // CH):
        tok0 = base + cch * CH
        pltpu.sync_copy(x2_hbm.at[pl.ds(tok0, CH)], rows_v)
        pltpu.sync_copy(pos_hbm.at[0, pl.ds(tok0, CH)], idx0_v)
        pltpu.sync_copy(pos_hbm.at[1, pl.ds(tok0, CH)], idx1_v)
        pltpu.sync_copy(rows_v, xs_hbm.at[idx0_v])
        pltpu.sync_copy(rows_v, xs_hbm.at[idx1_v])


def _gmm_body(te_ref, tv_ref, xs_ref, w1_ref, w2_ref, y_ref,
              w1b_ref, w2b_ref):
    i = pl.program_id(0)
    new_exp = jnp.logical_or(i == 0,
                             te_ref[i] != te_ref[jnp.maximum(i - 1, 0)])

    @pl.when(new_exp)
    def _cache_cast():
        w1b_ref[...] = w1_ref[0].astype(jnp.bfloat16)
        w2b_ref[...] = w2_ref[0].astype(jnp.bfloat16)

    @pl.when(tv_ref[i] == 1)
    def _compute():
        xb = _unpack_bf16(xs_ref[...])
        h = jax.lax.dot_general(xb, w1b_ref[...],
                                (((1,), (1,)), ((), ())),
                                preferred_element_type=jnp.float32)
        h = (h * jax.nn.sigmoid(h)).astype(jnp.bfloat16)
        y = jax.lax.dot_general(h, w2b_ref[...],
                                (((1,), (1,)), ((), ())),
                                preferred_element_type=jnp.float32)
        y_ref[...] = _pack_bf16(y)


def _sc_gather_body(ys_hbm, posf_hbm, yg_hbm, rows_v, idx_v, sem):
    wid = lax.axis_index("s") * NC + lax.axis_index("c")
    base = wid * ROW_PER_W
    for cch in range(ROW_PER_W // CH):
        r0 = base + cch * CH
        pltpu.sync_copy(posf_hbm.at[pl.ds(r0, CH)], idx_v)
        pltpu.async_copy(ys_hbm.at[idx_v], rows_v, sem).wait()
        pltpu.sync_copy(rows_v, yg_hbm.at[pl.ds(r0, CH)])


def _combine_body(hid_ref, y0_ref, y1_ref, ln2_ref, wg_ref, wse_ref, wsd_ref,
                  out_ref):
    x = hid_ref[...]
    x2 = _rms_x2(x, ln2_ref[...])
    logits = jax.lax.dot_general(x2, wg_ref[...], (((1,), (1,)), ((), ())),
                                 preferred_element_type=jnp.float32)
    probs = jax.nn.softmax(logits, axis=-1)
    iota = jax.lax.broadcasted_iota(jnp.int32, probs.shape, 1)
    m1 = jnp.max(probs, axis=-1, keepdims=True)
    i1 = jnp.min(jnp.where(probs == m1, iota, E), axis=-1, keepdims=True)
    probs_m = jnp.where(iota == i1, -jnp.inf, probs)
    m2 = jnp.max(probs_m, axis=-1, keepdims=True)
    w0 = m1 / (m1 + m2)
    w1c = m2 / (m1 + m2)

    gu = jax.lax.dot_general(x2.astype(jnp.bfloat16),
                             wse_ref[...].astype(jnp.bfloat16),
                             (((1,), (1,)), ((), ())),
                             preferred_element_type=jnp.float32)
    gate = gu[:, :DFF]
    up = gu[:, DFF:]
    act = (gate * jax.nn.sigmoid(gate) * up).astype(jnp.bfloat16)
    shared = jax.lax.dot_general(act, wsd_ref[...].astype(jnp.bfloat16),
                                 (((1,), (1,)), ((), ())),
                                 preferred_element_type=jnp.float32)
    y0 = _unpack_bf16(y0_ref[0]).astype(jnp.float32)
    y1 = _unpack_bf16(y1_ref[0]).astype(jnp.float32)
    out_ref[...] = x + w0 * y0 + w1c * y1 + shared


def kernel(hidden_states, positions, kv_cache, attn_metadata, ln1_w, ln2_w,
           Wq, Wkv, Wo, Wg, w1, w2, Wse, Wsd):
    ln2 = ln2_w.reshape(1, HID)

    x2, topi, counts = pl.pallas_call(
        _count_body,
        grid=(T // TBR,),
        in_specs=[
            pl.BlockSpec((TBR, HID), lambda t: (t, 0)),
            pl.BlockSpec((1, HID), lambda t: (0, 0)),
            pl.BlockSpec((E, HID), lambda t: (0, 0)),
        ],
        out_specs=[
            pl.BlockSpec((TBR, HIDW), lambda t: (t, 0)),
            pl.BlockSpec((2, TBR), lambda t: (0, t)),
            pl.BlockSpec((E, 1), lambda t: (0, 0)),
        ],
        out_shape=[
            jax.ShapeDtypeStruct((T, HIDW), jnp.int32),
            jax.ShapeDtypeStruct((2, T), jnp.int32),
            jax.ShapeDtypeStruct((E, 1), jnp.float32),
        ],
        compiler_params=pltpu.CompilerParams(
            dimension_semantics=("arbitrary",)),
    )(hidden_states, ln2, Wg)

    pos, te, tv = pl.pallas_call(
        _pos_body,
        grid=(T // TBR,),
        in_specs=[
            pl.BlockSpec((2, TBR), lambda t: (0, t)),
            pl.BlockSpec((E, 1), lambda t: (0, 0)),
        ],
        out_specs=[
            pl.BlockSpec((2, TBR), lambda t: (0, t)),
            pl.BlockSpec((1, NT), lambda t: (0, 0)),
            pl.BlockSpec((1, NT), lambda t: (0, 0)),
        ],
        out_shape=[
            jax.ShapeDtypeStruct((2, T), jnp.int32),
            jax.ShapeDtypeStruct((1, NT), jnp.int32),
            jax.ShapeDtypeStruct((1, NT), jnp.int32),
        ],
        scratch_shapes=[
            pltpu.VMEM((E, 1), jnp.float32),
            pltpu.VMEM((E, 1), jnp.float32),
        ],
        compiler_params=pltpu.CompilerParams(
            dimension_semantics=("arbitrary",)),
    )(topi, counts)

    mesh = plsc.VectorSubcoreMesh(core_axis_name="c", subcore_axis_name="s")

    scatter_k = functools.partial(
        pl.kernel, mesh=mesh,
        out_type=jax.ShapeDtypeStruct((NTP, HIDW), jnp.int32),
        scratch_types=[
            pltpu.VMEM((CH, HIDW), jnp.int32),
            pltpu.VMEM((CH,), jnp.int32),
            pltpu.VMEM((CH,), jnp.int32),
        ],
    )(_sc_scatter_body)
    xs = scatter_k(x2, pos)

    y = pl.pallas_call(
        _gmm_body,
        grid_spec=pltpu.PrefetchScalarGridSpec(
            num_scalar_prefetch=2,
            grid=(NT,),
            in_specs=[
                pl.BlockSpec((TBM, HIDW), lambda i, te, tv: (i, 0)),
                pl.BlockSpec((1, DFF, HID), lambda i, te, tv: (te[i], 0, 0)),
                pl.BlockSpec((1, HID, DFF), lambda i, te, tv: (te[i], 0, 0)),
            ],
            out_specs=pl.BlockSpec((TBM, HIDW), lambda i, te, tv: (i, 0)),
            scratch_shapes=[
                pltpu.VMEM((DFF, HID), jnp.bfloat16),
                pltpu.VMEM((HID, DFF), jnp.bfloat16),
            ],
        ),
        out_shape=jax.ShapeDtypeStruct((NTP, HIDW), jnp.int32),
        compiler_params=pltpu.CompilerParams(
            dimension_semantics=("arbitrary",)),
    )(te.reshape(NT), tv.reshape(NT), xs, w1, w2)

    gather_k = functools.partial(
        pl.kernel, mesh=mesh,
        out_type=jax.ShapeDtypeStruct((NSLOT, HIDW), jnp.int32),
        scratch_types=[
            pltpu.VMEM((CH, HIDW), jnp.int32),
            pltpu.VMEM((CH,), jnp.int32),
            pltpu.SemaphoreType.DMA,
        ],
    )(_sc_gather_body)
    yg = gather_k(y, pos.reshape(NSLOT)).reshape(2, T, HIDW)

    out = pl.pallas_call(
        _combine_body,
        grid=(T // TBC,),
        in_specs=[
            pl.BlockSpec((TBC, HID), lambda t: (t, 0)),
            pl.BlockSpec((1, TBC, HIDW), lambda t: (0, t, 0)),
            pl.BlockSpec((1, TBC, HIDW), lambda t: (1, t, 0)),
            pl.BlockSpec((1, HID), lambda t: (0, 0)),
            pl.BlockSpec((E, HID), lambda t: (0, 0)),
            pl.BlockSpec((2 * DFF, HID), lambda t: (0, 0)),
            pl.BlockSpec((HID, DFF), lambda t: (0, 0)),
        ],
        out_specs=pl.BlockSpec((TBC, HID), lambda t: (t, 0)),
        out_shape=jax.ShapeDtypeStruct((T, HID), jnp.float32),
    )(hidden_states, yg, yg, ln2, Wg, Wse, Wsd)

    return out


# speedup vs baseline: 5.8653x; 1.0571x over previous
import functools

import jax
import jax.numpy as jnp
from jax import lax
from jax.experimental import pallas as pl
from jax.experimental.pallas import tpu as pltpu
from jax.experimental.pallas import tpu_sc as plsc

T = 2048
HID = 2048
E = 8
DFF = 768
EPS = 1e-6

TBR = 512
TBM = 512
NSLOT = 2 * T
NT = NSLOT // TBM + E
NTP = NT * TBM
TBC = 512
HIDW = HID // 2

NC = 2
NS = 16
NW = NC * NS
TOK_PER_W = T // NW
ROW_PER_W = NSLOT // NW
CH = 64


def _rms_x2(x, ln2):
    var = jnp.mean(x * x, axis=-1, keepdims=True)
    return x * jax.lax.rsqrt(var + EPS) * ln2


def _pack_bf16(x):
    c = x.shape[1] // 2
    lo = jax.lax.bitcast_convert_type(x[:, :c].astype(jnp.bfloat16), jnp.int16)
    hi = jax.lax.bitcast_convert_type(x[:, c:].astype(jnp.bfloat16), jnp.int16)
    lo32 = lo.astype(jnp.int32) & 0xFFFF
    hi32 = hi.astype(jnp.int32) & 0xFFFF
    return (hi32 << 16) | lo32


def _unpack_bf16(w):
    lo = jax.lax.bitcast_convert_type((w & 0xFFFF).astype(jnp.int16),
                                      jnp.bfloat16)
    hi = jax.lax.bitcast_convert_type(
        ((w >> 16) & 0xFFFF).astype(jnp.int16), jnp.bfloat16)
    return jnp.concatenate([lo, hi], axis=1)


def _count_body(hid_ref, ln2_ref, wg_ref, x2_ref, topi_ref, counts_ref):
    t = pl.program_id(0)
    x = hid_ref[...]
    x2 = _rms_x2(x, ln2_ref[...])
    x2_ref[...] = _pack_bf16(x2)

    logits = jax.lax.dot_general(wg_ref[...], x2, (((1,), (1,)), ((), ())),
                                 preferred_element_type=jnp.float32)
    probs = jax.nn.softmax(logits, axis=0)
    iota_e = jax.lax.broadcasted_iota(jnp.int32, (E, TBR), 0)
    m1 = jnp.max(probs, axis=0, keepdims=True)
    i1 = jnp.min(jnp.where(probs == m1, iota_e, E), axis=0, keepdims=True)
    sel1 = (iota_e == i1).astype(jnp.float32)
    probs_m = jnp.where(iota_e == i1, -jnp.inf, probs)
    m2 = jnp.max(probs_m, axis=0, keepdims=True)
    i2 = jnp.min(jnp.where(probs_m == m2, iota_e, E), axis=0, keepdims=True)
    sel2 = (iota_e == i2).astype(jnp.float32)
    topi_ref[...] = jnp.concatenate([i1, i2], axis=0)

    @pl.when(t == 0)
    def _init_counts():
        counts_ref[...] = jnp.zeros_like(counts_ref)

    counts_ref[...] += (jnp.sum(sel1, axis=1, keepdims=True)
                        + jnp.sum(sel2, axis=1, keepdims=True))


def _pos_body(topi_ref, counts_ref, pos_ref, te_ref, tv_ref,
              run_ref, off_ref):
    t = pl.program_id(0)
    iota_e = jax.lax.broadcasted_iota(jnp.int32, (E, TBR), 0)
    ti = topi_ref[...]
    sel1 = (iota_e == ti[0:1, :]).astype(jnp.float32)
    sel2 = (iota_e == ti[1:2, :]).astype(jnp.float32)

    @pl.when(t == 0)
    def _offsets():
        c = counts_ref[...]
        pc = jnp.floor((c + (TBM - 1)) / TBM) * TBM
        a = jax.lax.broadcasted_iota(jnp.int32, (E, E), 0)
        b = jax.lax.broadcasted_iota(jnp.int32, (E, E), 1)
        lte = (b < a).astype(jnp.float32)
        off = jax.lax.dot_general(lte, pc, (((1,), (0,)), ((), ())),
                                  preferred_element_type=jnp.float32)
        off_ref[...] = off
        run_ref[...] = jnp.zeros_like(run_ref)
        s = (jax.lax.broadcasted_iota(jnp.int32, (E, NT), 1) * TBM
             ).astype(jnp.float32)
        offb = jnp.broadcast_to(off, (E, NT))
        pcb = jnp.broadcast_to(pc, (E, NT))
        mask = ((s >= offb) & (s < offb + pcb)).astype(jnp.int32)
        eio = jax.lax.broadcasted_iota(jnp.int32, (E, NT), 0)
        anym = jnp.max(mask, axis=0, keepdims=True)
        te = jnp.sum(mask * eio, axis=0, keepdims=True)
        te_ref[...] = jnp.where(anym == 1, te, E - 1)
        tv_ref[...] = anym

    run = run_ref[...]
    off = off_ref[...]
    r = jax.lax.broadcasted_iota(jnp.int32, (TBR, TBR), 0)
    c = jax.lax.broadcasted_iota(jnp.int32, (TBR, TBR), 1)
    lt = (r < c).astype(jnp.float32)
    lc1 = jax.lax.dot_general(sel1, lt, (((1,), (0,)), ((), ())),
                              preferred_element_type=jnp.float32)
    pos0 = jnp.sum(sel1 * (off + run + lc1), axis=0, keepdims=True)
    run1 = run + jnp.sum(sel1, axis=1, keepdims=True)
    lc2 = jax.lax.dot_general(sel2, lt, (((1,), (0,)), ((), ())),
                              preferred_element_type=jnp.float32)
    pos1 = jnp.sum(sel2 * (off + run1 + lc2), axis=0, keepdims=True)
    run_ref[...] = run1 + jnp.sum(sel2, axis=1, keepdims=True)
    pos_ref[...] = jnp.concatenate([pos0, pos1], axis=0).astype(jnp.int32)


def _sc_scatter_body(x2_hbm, pos_hbm, xs_hbm, rows_v, idx0_v, idx1_v):
    wid = lax.axis_index("s") * NC + lax.axis_index("c")
    base = wid * TOK_PER_W
    for cch in range(TOK_PER_W // CH):
        tok0 = base + cch * CH
        pltpu.sync_copy(x2_hbm.at[pl.ds(tok0, CH)], rows_v)
        pltpu.sync_copy(pos_hbm.at[0, pl.ds(tok0, CH)], idx0_v)
        pltpu.sync_copy(pos_hbm.at[1, pl.ds(tok0, CH)], idx1_v)
        pltpu.sync_copy(rows_v, xs_hbm.at[idx0_v])
        pltpu.sync_copy(rows_v, xs_hbm.at[idx1_v])


def _gmm_body(te_ref, tv_ref, xs_ref, w1_ref, w2_ref, y_ref,
              w1b_ref, w2b_ref):
    i = pl.program_id(0)
    new_exp = jnp.logical_or(i == 0,
                             te_ref[i] != te_ref[jnp.maximum(i - 1, 0)])

    @pl.when(new_exp)
    def _cache_cast():
        w1b_ref[...] = w1_ref[0].astype(jnp.bfloat16)
        w2b_ref[...] = w2_ref[0].astype(jnp.bfloat16)

    @pl.when(tv_ref[i] == 1)
    def _compute():
        xb = _unpack_bf16(xs_ref[...])
        h = jax.lax.dot_general(xb, w1b_ref[...],
                                (((1,), (1,)), ((), ())),
                                preferred_element_type=jnp.float32)
        h = (h * jax.nn.sigmoid(h)).astype(jnp.bfloat16)
        y = jax.lax.dot_general(h, w2b_ref[...],
                                (((1,), (1,)), ((), ())),
                                preferred_element_type=jnp.float32)
        y_ref[...] = _pack_bf16(y)


def _sc_gather_body(ys_hbm, posf_hbm, yg_hbm, rows_v, idx_v, sem):
    wid = lax.axis_index("s") * NC + lax.axis_index("c")
    base = wid * ROW_PER_W
    for cch in range(ROW_PER_W // CH):
        r0 = base + cch * CH
        pltpu.sync_copy(posf_hbm.at[pl.ds(r0, CH)], idx_v)
        pltpu.async_copy(ys_hbm.at[idx_v], rows_v, sem).wait()
        pltpu.sync_copy(rows_v, yg_hbm.at[pl.ds(r0, CH)])


def _combine_body(hid_ref, y0_ref, y1_ref, ln2_ref, wg_ref, wse_ref, wsd_ref,
                  out_ref):
    x = hid_ref[...]
    x2 = _rms_x2(x, ln2_ref[...])
    logits = jax.lax.dot_general(x2, wg_ref[...], (((1,), (1,)), ((), ())),
                                 preferred_element_type=jnp.float32)
    probs = jax.nn.softmax(logits, axis=-1)
    iota = jax.lax.broadcasted_iota(jnp.int32, probs.shape, 1)
    m1 = jnp.max(probs, axis=-1, keepdims=True)
    i1 = jnp.min(jnp.where(probs == m1, iota, E), axis=-1, keepdims=True)
    probs_m = jnp.where(iota == i1, -jnp.inf, probs)
    m2 = jnp.max(probs_m, axis=-1, keepdims=True)
    w0 = m1 / (m1 + m2)
    w1c = m2 / (m1 + m2)

    gu = jax.lax.dot_general(x2.astype(jnp.bfloat16),
                             wse_ref[...].astype(jnp.bfloat16),
                             (((1,), (1,)), ((), ())),
                             preferred_element_type=jnp.float32)
    gate = gu[:, :DFF]
    up = gu[:, DFF:]
    act = (gate * jax.nn.sigmoid(gate) * up).astype(jnp.bfloat16)
    shared = jax.lax.dot_general(act, wsd_ref[...].astype(jnp.bfloat16),
                                 (((1,), (1,)), ((), ())),
                                 preferred_element_type=jnp.float32)
    y0 = _unpack_bf16(y0_ref[0]).astype(jnp.float32)
    y1 = _unpack_bf16(y1_ref[0]).astype(jnp.float32)
    out_ref[...] = x + w0 * y0 + w1c * y1 + shared


def kernel(hidden_states, positions, kv_cache, attn_metadata, ln1_w, ln2_w,
           Wq, Wkv, Wo, Wg, w1, w2, Wse, Wsd):
    ln2 = ln2_w.reshape(1, HID)

    x2, topi, counts = pl.pallas_call(
        _count_body,
        grid=(T // TBR,),
        in_specs=[
            pl.BlockSpec((TBR, HID), lambda t: (t, 0)),
            pl.BlockSpec((1, HID), lambda t: (0, 0)),
            pl.BlockSpec((E, HID), lambda t: (0, 0)),
        ],
        out_specs=[
            pl.BlockSpec((TBR, HIDW), lambda t: (t, 0)),
            pl.BlockSpec((2, TBR), lambda t: (0, t)),
            pl.BlockSpec((E, 1), lambda t: (0, 0)),
        ],
        out_shape=[
            jax.ShapeDtypeStruct((T, HIDW), jnp.int32),
            jax.ShapeDtypeStruct((2, T), jnp.int32),
            jax.ShapeDtypeStruct((E, 1), jnp.float32),
        ],
        compiler_params=pltpu.CompilerParams(
            dimension_semantics=("arbitrary",)),
    )(hidden_states, ln2, Wg)

    pos, te, tv = pl.pallas_call(
        _pos_body,
        grid=(T // TBR,),
        in_specs=[
            pl.BlockSpec((2, TBR), lambda t: (0, t)),
            pl.BlockSpec((E, 1), lambda t: (0, 0)),
        ],
        out_specs=[
            pl.BlockSpec((2, TBR), lambda t: (0, t)),
            pl.BlockSpec((1, NT), lambda t: (0, 0)),
            pl.BlockSpec((1, NT), lambda t: (0, 0)),
        ],
        out_shape=[
            jax.ShapeDtypeStruct((2, T), jnp.int32),
            jax.ShapeDtypeStruct((1, NT), jnp.int32),
            jax.ShapeDtypeStruct((1, NT), jnp.int32),
        ],
        scratch_shapes=[
            pltpu.VMEM((E, 1), jnp.float32),
            pltpu.VMEM((E, 1), jnp.float32),
        ],
        compiler_params=pltpu.CompilerParams(
            dimension_semantics=("arbitrary",)),
    )(topi, counts)

    mesh = plsc.VectorSubcoreMesh(core_axis_name="c", subcore_axis_name="s")

    scatter_k = functools.partial(
        pl.kernel, mesh=mesh,
        out_type=jax.ShapeDtypeStruct((NTP, HIDW), jnp.int32),
        scratch_types=[
            pltpu.VMEM((CH, HIDW), jnp.int32),
            pltpu.VMEM((CH,), jnp.int32),
            pltpu.VMEM((CH,), jnp.int32),
        ],
    )(_sc_scatter_body)
    xs = scatter_k(x2, pos)

    y = pl.pallas_call(
        _gmm_body,
        grid_spec=pltpu.PrefetchScalarGridSpec(
            num_scalar_prefetch=2,
            grid=(NT,),
            in_specs=[
                pl.BlockSpec((TBM, HIDW), lambda i, te, tv: (i, 0)),
                pl.BlockSpec((1, DFF, HID), lambda i, te, tv: (te[i], 0, 0)),
                pl.BlockSpec((1, HID, DFF), lambda i, te, tv: (te[i], 0, 0)),
            ],
            out_specs=pl.BlockSpec((TBM, HIDW), lambda i, te, tv: (i, 0)),
            scratch_shapes=[
                pltpu.VMEM((DFF, HID), jnp.bfloat16),
                pltpu.VMEM((HID, DFF), jnp.bfloat16),
            ],
        ),
        out_shape=jax.ShapeDtypeStruct((NTP, HIDW), jnp.int32),
        compiler_params=pltpu.CompilerParams(
            dimension_semantics=("arbitrary",)),
    )(te.reshape(NT), tv.reshape(NT), xs, w1, w2)

    gather_k = functools.partial(
        pl.kernel, mesh=mesh,
        out_type=jax.ShapeDtypeStruct((NSLOT, HIDW), jnp.int32),
        scratch_types=[
            pltpu.VMEM((CH, HIDW), jnp.int32),
            pltpu.VMEM((CH,), jnp.int32),
            pltpu.SemaphoreType.DMA,
        ],
    )(_sc_gather_body)
    yg = gather_k(y, pos.reshape(NSLOT)).reshape(2, T, HIDW)

    out = pl.pallas_call(
        _combine_body,
        grid=(T // TBC,),
        in_specs=[
            pl.BlockSpec((TBC, HID), lambda t: (t, 0)),
            pl.BlockSpec((1, TBC, HIDW), lambda t: (0, t, 0)),
            pl.BlockSpec((1, TBC, HIDW), lambda t: (1, t, 0)),
            pl.BlockSpec((1, HID), lambda t: (0, 0)),
            pl.BlockSpec((E, HID), lambda t: (0, 0)),
            pl.BlockSpec((2 * DFF, HID), lambda t: (0, 0)),
            pl.BlockSpec((HID, DFF), lambda t: (0, 0)),
        ],
        out_specs=pl.BlockSpec((TBC, HID), lambda t: (t, 0)),
        out_shape=jax.ShapeDtypeStruct((T, HID), jnp.float32),
    )(hidden_states, yg, yg, ln2, Wg, Wse, Wsd)

    return out
